# Initial kernel scaffold; baseline (speedup 1.0000x reference)
import jax
import jax.numpy as jnp
from jax.experimental import pallas as pl


def kernel(x, edge_index, W1, b1, W2, b2):
    raise NotImplementedError("write your pallas kernel here")


# speedup vs baseline: 41.6598x; 41.6598x over previous
import functools

import jax
import jax.numpy as jnp
from jax import lax
from jax.experimental import pallas as pl
from jax.experimental.pallas import tpu as pltpu
from jax.experimental.pallas import tpu_sc as plsc

NC = 2
NS = 16
NW = NC * NS
CH = 128
NBUF = 4


def _sc_mesh():
    return plsc.VectorSubcoreMesh(
        core_axis_name="c", subcore_axis_name="s",
        num_cores=NC, num_subcores=NS)


def _sc_params():
    return pltpu.CompilerParams(use_tc_tiling_on_sc=False)


def _deg_kernel(npad, epw):
    zp = npad // NS
    nch = epw // CH

    @functools.partial(
        pl.kernel,
        out_type=jax.ShapeDtypeStruct((NC, npad), jnp.float32),
        mesh=_sc_mesh(),
        compiler_params=_sc_params(),
        scratch_types=[
            pltpu.VMEM((NBUF, CH), jnp.int32),
            pltpu.VMEM((CH,), jnp.float32),
            pltpu.VMEM((zp,), jnp.float32),
            pltpu.VMEM_SHARED((npad,), jnp.float32),
            pltpu.SemaphoreType.DMA((NBUF,)),
            pltpu.SemaphoreType.DMA((2,)),
        ],
    )
    def k(dst_hbm, ones_hbm, zeros_hbm, out_hbm, dsti, onesb, stage, degsh,
          si, ss):
        cid = lax.axis_index("c")
        sid = lax.axis_index("s")
        wid = sid * NC + cid
        pltpu.sync_copy(ones_hbm, onesb)
        pltpu.sync_copy(zeros_hbm, stage)
        pltpu.sync_copy(stage, degsh.at[pl.ds(sid * zp, zp)])
        plsc.subcore_barrier()

        def issue_i(c, b):
            pltpu.async_copy(dst_hbm.at[wid, pl.ds(c * CH, CH)],
                             dsti.at[b], si.at[b])

        def wait_i(c, b):
            pltpu.make_async_copy(dst_hbm.at[wid, pl.ds(c * CH, CH)],
                                  dsti.at[b], si.at[b]).wait()

        def issue_s(c, b):
            pltpu.async_copy(onesb, degsh.at[dsti.at[b]], ss.at[b % 2],
                             add=True)

        def wait_s(c, b):
            pltpu.make_async_copy(onesb, degsh.at[dsti.at[b]],
                                  ss.at[b % 2]).wait()

        def body(c, b, w_s2, i_i2):
            if w_s2:
                wait_s(c - 2, (b - 2) % NBUF)
            wait_i(c, b)
            issue_s(c, b)
            if i_i2:
                issue_i(c + 2, (b + 2) % NBUF)

        issue_i(0, 0)
        issue_i(1, 1)
        body(0, 0, False, True)
        body(1, 1, False, True)
        body(2, 2, True, True)
        body(3, 3, True, True)

        def grp(g, carry):
            for b in range(NBUF):
                body(NBUF * g + b, b, True, True)
            return carry

        lax.fori_loop(1, nch // NBUF - 1, grp, 0)
        for b in range(NBUF):
            body(nch - NBUF + b, b, True, b < 2)
        wait_s(nch - 2, (nch - 2) % NBUF)
        wait_s(nch - 1, (nch - 1) % NBUF)

        plsc.subcore_barrier()
        pltpu.sync_copy(degsh.at[pl.ds(sid * zp, zp)], stage)
        pltpu.sync_copy(stage, out_hbm.at[cid, pl.ds(sid * zp, zp)])

    return k


def _agg_kernel(npad, epw, h):
    zp = npad // NS
    nch = epw // CH
    scalar = h is None
    rows_shape = (2, CH) if scalar else (2, CH, h)
    acc_shape = (npad,) if scalar else (npad, h)
    out_shape = (NC, npad) if scalar else (NC, npad, h)
    stage_shape = (zp,) if scalar else (zp, h)

    @functools.partial(
        pl.kernel,
        out_type=jax.ShapeDtypeStruct(out_shape, jnp.float32),
        mesh=_sc_mesh(),
        compiler_params=_sc_params(),
        scratch_types=[
            pltpu.VMEM((NBUF, CH), jnp.int32),
            pltpu.VMEM((NBUF, CH), jnp.int32),
            pltpu.VMEM(rows_shape, jnp.float32),
            pltpu.VMEM(stage_shape, jnp.float32),
            pltpu.VMEM_SHARED(acc_shape, jnp.float32),
            pltpu.SemaphoreType.DMA((NBUF,)),
            pltpu.SemaphoreType.DMA((NBUF,)),
            pltpu.SemaphoreType.DMA((2,)),
            pltpu.SemaphoreType.DMA((2,)),
        ],
    )
    def k(g_hbm, src_hbm, dst_hbm, zeros_hbm, out_hbm,
          srci, dsti, rows, stage, accsh, sis, sid_, sg, ss):
        cid = lax.axis_index("c")
        sid = lax.axis_index("s")
        wid = sid * NC + cid
        pltpu.sync_copy(zeros_hbm, stage)
        pltpu.sync_copy(stage, accsh.at[pl.ds(sid * zp, zp)])
        plsc.subcore_barrier()

        def issue_i(c, b):
            pltpu.async_copy(src_hbm.at[wid, pl.ds(c * CH, CH)],
                             srci.at[b], sis.at[b])
            pltpu.async_copy(dst_hbm.at[wid, pl.ds(c * CH, CH)],
                             dsti.at[b], sid_.at[b])

        def wait_i(c, b):
            pltpu.make_async_copy(src_hbm.at[wid, pl.ds(c * CH, CH)],
                                  srci.at[b], sis.at[b]).wait()
            pltpu.make_async_copy(dst_hbm.at[wid, pl.ds(c * CH, CH)],
                                  dsti.at[b], sid_.at[b]).wait()

        def issue_g(c, b):
            pltpu.async_copy(g_hbm.at[srci.at[b]], rows.at[b % 2],
                             sg.at[b % 2])

        def wait_g(c, b):
            pltpu.make_async_copy(g_hbm.at[srci.at[b]], rows.at[b % 2],
                                  sg.at[b % 2]).wait()

        def issue_s(c, b):
            pltpu.async_copy(rows.at[b % 2], accsh.at[dsti.at[b]],
                             ss.at[b % 2], add=True)

        def wait_s(c, b):
            pltpu.make_async_copy(rows.at[b % 2], accsh.at[dsti.at[b]],
                                  ss.at[b % 2]).wait()

        def body(c, b, w_s2, w_g1, i_i2):
            if w_s2:
                wait_s(c - 2, (b - 2) % NBUF)
            wait_i(c, b)
            issue_g(c, b)
            if i_i2:
                issue_i(c + 2, (b + 2) % NBUF)
            if w_g1:
                wait_g(c - 1, (b - 1) % NBUF)
                issue_s(c - 1, (b - 1) % NBUF)

        issue_i(0, 0)
        issue_i(1, 1)
        body(0, 0, False, False, True)
        body(1, 1, False, True, True)
        body(2, 2, True, True, True)
        body(3, 3, True, True, True)

        def grp(g, carry):
            for b in range(NBUF):
                body(NBUF * g + b, b, True, True, True)
            return carry

        lax.fori_loop(1, nch // NBUF - 1, grp, 0)
        for b in range(NBUF):
            body(nch - NBUF + b, b, True, True, b < 2)
        wait_g(nch - 1, (nch - 1) % NBUF)
        issue_s(nch - 1, (nch - 1) % NBUF)
        wait_s(nch - 2, (nch - 2) % NBUF)
        wait_s(nch - 1, (nch - 1) % NBUF)

        plsc.subcore_barrier()
        pltpu.sync_copy(accsh.at[pl.ds(sid * zp, zp)], stage)
        pltpu.sync_copy(stage, out_hbm.at[cid, pl.ds(sid * zp, zp)])

    return k


def _tc1_body(degp_ref, x_ref, w1_ref, g1_ref, dis_ref):
    deg = degp_ref[0, :] + degp_ref[1, :] + 1.0
    dis = lax.rsqrt(deg)
    hm = jnp.dot(x_ref[...], w1_ref[...], preferred_element_type=jnp.float32)
    g1_ref[...] = hm * dis[:, None]
    dis_ref[...] = dis[:, None]


def _tc2_body(accp_ref, g1_ref, dis_ref, b1_ref, w2r_ref, g2_ref):
    a = accp_ref[0] + accp_ref[1] + g1_ref[...]
    z = jnp.maximum(dis_ref[...] * a + b1_ref[...], 0.0)
    h2 = jnp.sum(z * w2r_ref[...], axis=1, keepdims=True)
    g2_ref[...] = dis_ref[...] * h2


def _tc3_body(aggp_ref, g2_ref, dis_ref, b2_ref, out_ref):
    s = (aggp_ref[0] + aggp_ref[1])[:, None] + g2_ref[...]
    out_ref[...] = jax.nn.sigmoid(dis_ref[...] * s + b2_ref[...])


def kernel(x, edge_index, W1, b1, W2, b2):
    n, d = x.shape
    h = W1.shape[1]
    e = edge_index.shape[1]

    blk = 1024
    npad = ((n + 1 + blk - 1) // blk) * blk
    grid = npad // blk
    epw = ((e + NW * NBUF * CH - 1) // (NW * NBUF * CH)) * NBUF * CH
    zp = npad // NS

    xp = jnp.pad(x, ((0, npad - n), (0, 0)))
    pad_e = NW * epw - e
    srcp = jnp.concatenate(
        [edge_index[0], jnp.zeros((pad_e,), jnp.int32)]).reshape(NW, epw)
    dstp = jnp.concatenate(
        [edge_index[1], jnp.full((pad_e,), n, jnp.int32)]).reshape(NW, epw)

    ones_ch = jnp.ones((CH,), jnp.float32)
    zeros1 = jnp.zeros((zp,), jnp.float32)
    zeros2 = jnp.zeros((zp, h), jnp.float32)

    degp = _deg_kernel(npad, epw)(dstp, ones_ch, zeros1)

    g1, dis = pl.pallas_call(
        _tc1_body,
        grid=(grid,),
        in_specs=[
            pl.BlockSpec((NC, blk), lambda i: (0, i)),
            pl.BlockSpec((blk, d), lambda i: (i, 0)),
            pl.BlockSpec((d, h), lambda i: (0, 0)),
        ],
        out_specs=[
            pl.BlockSpec((blk, h), lambda i: (i, 0)),
            pl.BlockSpec((blk, 1), lambda i: (i, 0)),
        ],
        out_shape=[
            jax.ShapeDtypeStruct((npad, h), jnp.float32),
            jax.ShapeDtypeStruct((npad, 1), jnp.float32),
        ],
    )(degp, xp, W1)

    accp = _agg_kernel(npad, epw, h)(g1, srcp, dstp, zeros2)

    g2 = pl.pallas_call(
        _tc2_body,
        grid=(grid,),
        in_specs=[
            pl.BlockSpec((NC, blk, h), lambda i: (0, i, 0)),
            pl.BlockSpec((blk, h), lambda i: (i, 0)),
            pl.BlockSpec((blk, 1), lambda i: (i, 0)),
            pl.BlockSpec((1, h), lambda i: (0, 0)),
            pl.BlockSpec((1, h), lambda i: (0, 0)),
        ],
        out_specs=pl.BlockSpec((blk, 1), lambda i: (i, 0)),
        out_shape=jax.ShapeDtypeStruct((npad, 1), jnp.float32),
    )(accp, g1, dis, b1.reshape(1, h), W2.reshape(1, h))

    g2f = g2.reshape(npad)
    agg2p = _agg_kernel(npad, epw, None)(g2f, srcp, dstp, zeros1)

    out = pl.pallas_call(
        _tc3_body,
        grid=(grid,),
        in_specs=[
            pl.BlockSpec((NC, blk), lambda i: (0, i)),
            pl.BlockSpec((blk, 1), lambda i: (i, 0)),
            pl.BlockSpec((blk, 1), lambda i: (i, 0)),
            pl.BlockSpec((1, 1), lambda i: (0, 0)),
        ],
        out_specs=pl.BlockSpec((blk, 1), lambda i: (i, 0)),
        out_shape=jax.ShapeDtypeStruct((npad, 1), jnp.float32),
    )(agg2p, g2, dis, b2.reshape(1, 1))

    return out[:n, :]


# speedup vs baseline: 52.9317x; 1.2706x over previous
import functools

import jax
import jax.numpy as jnp
from jax import lax
from jax.experimental import pallas as pl
from jax.experimental.pallas import tpu as pltpu
from jax.experimental.pallas import tpu_sc as plsc

NC = 2
NS = 16
NW = NC * NS
CH = 128
NBUF = 4


def _sc_mesh():
    return plsc.VectorSubcoreMesh(
        core_axis_name="c", subcore_axis_name="s",
        num_cores=NC, num_subcores=NS)


def _sc_params():
    return pltpu.CompilerParams(use_tc_tiling_on_sc=False)


def _pipeline(nch, body, issue_i, epilogue):
    fg, rem = nch // NBUF, nch % NBUF
    issue_i(0, 0)
    issue_i(1, 1)
    body(0, 0, False, False, True)
    body(1, 1, False, True, True)
    body(2, 2, True, True, True)
    body(3, 3, True, True, True)
    steady_end = fg if rem else fg - 1

    def grp(g, carry):
        for b in range(NBUF):
            body(NBUF * g + b, b, True, True, True)
        return carry

    lax.fori_loop(1, steady_end, grp, 0)
    tail_cs = range(NBUF * steady_end, nch)
    for c in tail_cs:
        body(c, c % NBUF, True, True, c + 2 < nch)
    epilogue()


def _deg_kernel(npad, e):
    zp = npad // NS
    et = e // NW
    nch = et // CH
    tail = et - nch * CH

    @functools.partial(
        pl.kernel,
        out_type=jax.ShapeDtypeStruct((NC, npad), jnp.float32),
        mesh=_sc_mesh(),
        compiler_params=_sc_params(),
        scratch_types=[
            pltpu.VMEM((NBUF, CH), jnp.int32),
            pltpu.VMEM((16,), jnp.int32),
            pltpu.VMEM((CH,), jnp.float32),
            pltpu.VMEM((zp,), jnp.float32),
            pltpu.VMEM_SHARED((npad,), jnp.float32),
            pltpu.SemaphoreType.DMA((NBUF,)),
            pltpu.SemaphoreType.DMA((2,)),
        ],
    )
    def k(edge_hbm, ones_hbm, zeros_hbm, out_hbm, dsti, dstt, onesb, stage,
          degsh, si, ss):
        cid = lax.axis_index("c")
        sid = lax.axis_index("s")
        wid = sid * NC + cid
        base = wid * et
        pltpu.sync_copy(ones_hbm, onesb)
        pltpu.sync_copy(zeros_hbm, stage)
        pltpu.sync_copy(stage, degsh.at[pl.ds(sid * zp, zp)])
        plsc.subcore_barrier()

        def issue_i(c, b):
            pltpu.async_copy(edge_hbm.at[1, pl.ds(base + c * CH, CH)],
                             dsti.at[b], si.at[b])

        def wait_i(c, b):
            pltpu.make_async_copy(edge_hbm.at[1, pl.ds(base + c * CH, CH)],
                                  dsti.at[b], si.at[b]).wait()

        def issue_s(c, b):
            pltpu.async_copy(onesb, degsh.at[dsti.at[b]], ss.at[b % 2],
                             add=True)

        def wait_s(c, b):
            pltpu.make_async_copy(onesb, degsh.at[dsti.at[b]],
                                  ss.at[b % 2]).wait()

        def body(c, b, w_s2, w_g1, i_i2):
            if w_s2:
                wait_s(c - 2, (b - 2) % NBUF)
            wait_i(c, b)
            issue_s(c, b)
            if i_i2:
                issue_i(c + 2, (b + 2) % NBUF)

        def epilogue():
            wait_s(nch - 2, (nch - 2) % NBUF)
            wait_s(nch - 1, (nch - 1) % NBUF)

        _pipeline(nch, body, issue_i, epilogue)

        if tail:
            pltpu.sync_copy(edge_hbm.at[1, pl.ds(base + nch * CH, tail)], dstt)
            pltpu.sync_copy(onesb.at[pl.ds(0, tail)], degsh.at[dstt], add=True)

        plsc.subcore_barrier()
        pltpu.sync_copy(degsh.at[pl.ds(sid * zp, zp)], stage)
        pltpu.sync_copy(stage, out_hbm.at[cid, pl.ds(sid * zp, zp)])

    return k


def _agg_kernel(npad, e, h):
    zp = npad // NS
    et = e // NW
    nch = et // CH
    tail = et - nch * CH
    scalar = h is None
    rows_shape = (2, CH) if scalar else (2, CH, h)
    trows_shape = (16,) if scalar else (16, h)
    acc_shape = (npad,) if scalar else (npad, h)
    out_shape = (NC, npad) if scalar else (NC, npad, h)
    stage_shape = (zp,) if scalar else (zp, h)

    @functools.partial(
        pl.kernel,
        out_type=jax.ShapeDtypeStruct(out_shape, jnp.float32),
        mesh=_sc_mesh(),
        compiler_params=_sc_params(),
        scratch_types=[
            pltpu.VMEM((NBUF, CH), jnp.int32),
            pltpu.VMEM((NBUF, CH), jnp.int32),
            pltpu.VMEM((16,), jnp.int32),
            pltpu.VMEM((16,), jnp.int32),
            pltpu.VMEM(rows_shape, jnp.float32),
            pltpu.VMEM(trows_shape, jnp.float32),
            pltpu.VMEM(stage_shape, jnp.float32),
            pltpu.VMEM_SHARED(acc_shape, jnp.float32),
            pltpu.SemaphoreType.DMA((NBUF,)),
            pltpu.SemaphoreType.DMA((NBUF,)),
            pltpu.SemaphoreType.DMA((2,)),
            pltpu.SemaphoreType.DMA((2,)),
        ],
    )
    def k(g_hbm, edge_hbm, zeros_hbm, out_hbm,
          srci, dsti, srct, dstt, rows, rowst, stage, accsh, sis, sid_, sg, ss):
        cid = lax.axis_index("c")
        sid = lax.axis_index("s")
        wid = sid * NC + cid
        base = wid * et
        pltpu.sync_copy(zeros_hbm, stage)
        pltpu.sync_copy(stage, accsh.at[pl.ds(sid * zp, zp)])
        plsc.subcore_barrier()

        def issue_i(c, b):
            pltpu.async_copy(edge_hbm.at[0, pl.ds(base + c * CH, CH)],
                             srci.at[b], sis.at[b])
            pltpu.async_copy(edge_hbm.at[1, pl.ds(base + c * CH, CH)],
                             dsti.at[b], sid_.at[b])

        def wait_i(c, b):
            pltpu.make_async_copy(edge_hbm.at[0, pl.ds(base + c * CH, CH)],
                                  srci.at[b], sis.at[b]).wait()
            pltpu.make_async_copy(edge_hbm.at[1, pl.ds(base + c * CH, CH)],
                                  dsti.at[b], sid_.at[b]).wait()

        def issue_g(c, b):
            pltpu.async_copy(g_hbm.at[srci.at[b]], rows.at[b % 2],
                             sg.at[b % 2])

        def wait_g(c, b):
            pltpu.make_async_copy(g_hbm.at[srci.at[b]], rows.at[b % 2],
                                  sg.at[b % 2]).wait()

        def issue_s(c, b):
            pltpu.async_copy(rows.at[b % 2], accsh.at[dsti.at[b]],
                             ss.at[b % 2], add=True)

        def wait_s(c, b):
            pltpu.make_async_copy(rows.at[b % 2], accsh.at[dsti.at[b]],
                                  ss.at[b % 2]).wait()

        def body(c, b, w_s2, w_g1, i_i2):
            if w_s2:
                wait_s(c - 2, (b - 2) % NBUF)
            wait_i(c, b)
            issue_g(c, b)
            if i_i2:
                issue_i(c + 2, (b + 2) % NBUF)
            if w_g1:
                wait_g(c - 1, (b - 1) % NBUF)
                issue_s(c - 1, (b - 1) % NBUF)

        def epilogue():
            wait_g(nch - 1, (nch - 1) % NBUF)
            issue_s(nch - 1, (nch - 1) % NBUF)
            wait_s(nch - 2, (nch - 2) % NBUF)
            wait_s(nch - 1, (nch - 1) % NBUF)

        _pipeline(nch, body, issue_i, epilogue)

        if tail:
            pltpu.sync_copy(edge_hbm.at[0, pl.ds(base + nch * CH, tail)], srct)
            pltpu.sync_copy(edge_hbm.at[1, pl.ds(base + nch * CH, tail)], dstt)
            pltpu.sync_copy(g_hbm.at[srct], rowst)
            pltpu.sync_copy(rowst, accsh.at[dstt], add=True)

        plsc.subcore_barrier()
        pltpu.sync_copy(accsh.at[pl.ds(sid * zp, zp)], stage)
        pltpu.sync_copy(stage, out_hbm.at[cid, pl.ds(sid * zp, zp)])

    return k


def _tc1_body(degp_ref, x_ref, w1_ref, g1_ref, dis_ref):
    deg = degp_ref[0, :] + degp_ref[1, :] + 1.0
    dis = lax.rsqrt(deg)
    hm = jnp.dot(x_ref[...], w1_ref[...], preferred_element_type=jnp.float32)
    g1_ref[...] = hm * dis[:, None]
    dis_ref[...] = dis[:, None]


def _tc2_body(accp_ref, g1_ref, dis_ref, b1_ref, w2r_ref, g2_ref):
    a = accp_ref[0] + accp_ref[1] + g1_ref[...]
    z = jnp.maximum(dis_ref[...] * a + b1_ref[...], 0.0)
    h2 = jnp.sum(z * w2r_ref[...], axis=1, keepdims=True)
    g2_ref[...] = dis_ref[...] * h2


def _tc3_body(aggp_ref, g2_ref, dis_ref, b2_ref, out_ref):
    s = (aggp_ref[0] + aggp_ref[1])[:, None] + g2_ref[...]
    out_ref[...] = jax.nn.sigmoid(dis_ref[...] * s + b2_ref[...])


def kernel(x, edge_index, W1, b1, W2, b2):
    n, d = x.shape
    h = W1.shape[1]
    e = edge_index.shape[1]

    blk = 1024
    npad = ((n + 1 + blk - 1) // blk) * blk
    grid = npad // blk
    zp = npad // NS

    xp = jnp.pad(x, ((0, npad - n), (0, 0)))
    ones_ch = jnp.ones((CH,), jnp.float32)
    zeros1 = jnp.zeros((zp,), jnp.float32)
    zeros2 = jnp.zeros((zp, h), jnp.float32)

    degp = _deg_kernel(npad, e)(edge_index, ones_ch, zeros1)

    g1, dis = pl.pallas_call(
        _tc1_body,
        grid=(grid,),
        in_specs=[
            pl.BlockSpec((NC, blk), lambda i: (0, i)),
            pl.BlockSpec((blk, d), lambda i: (i, 0)),
            pl.BlockSpec((d, h), lambda i: (0, 0)),
        ],
        out_specs=[
            pl.BlockSpec((blk, h), lambda i: (i, 0)),
            pl.BlockSpec((blk, 1), lambda i: (i, 0)),
        ],
        out_shape=[
            jax.ShapeDtypeStruct((npad, h), jnp.float32),
            jax.ShapeDtypeStruct((npad, 1), jnp.float32),
        ],
    )(degp, xp, W1)

    accp = _agg_kernel(npad, e, h)(g1, edge_index, zeros2)

    g2 = pl.pallas_call(
        _tc2_body,
        grid=(grid,),
        in_specs=[
            pl.BlockSpec((NC, blk, h), lambda i: (0, i, 0)),
            pl.BlockSpec((blk, h), lambda i: (i, 0)),
            pl.BlockSpec((blk, 1), lambda i: (i, 0)),
            pl.BlockSpec((1, h), lambda i: (0, 0)),
            pl.BlockSpec((1, h), lambda i: (0, 0)),
        ],
        out_specs=pl.BlockSpec((blk, 1), lambda i: (i, 0)),
        out_shape=jax.ShapeDtypeStruct((npad, 1), jnp.float32),
    )(accp, g1, dis, b1.reshape(1, h), W2.reshape(1, h))

    g2f = g2.reshape(npad)
    agg2p = _agg_kernel(npad, e, None)(g2f, edge_index, zeros1)

    out = pl.pallas_call(
        _tc3_body,
        grid=(grid,),
        in_specs=[
            pl.BlockSpec((NC, blk), lambda i: (0, i)),
            pl.BlockSpec((blk, 1), lambda i: (i, 0)),
            pl.BlockSpec((blk, 1), lambda i: (i, 0)),
            pl.BlockSpec((1, 1), lambda i: (0, 0)),
        ],
        out_specs=pl.BlockSpec((blk, 1), lambda i: (i, 0)),
        out_shape=jax.ShapeDtypeStruct((npad, 1), jnp.float32),
    )(agg2p, g2, dis, b2.reshape(1, 1))

    return out[:n, :]


# speedup vs baseline: 67.0148x; 1.2661x over previous
import functools

import jax
import jax.numpy as jnp
from jax import lax
from jax.experimental import pallas as pl
from jax.experimental.pallas import tpu as pltpu
from jax.experimental.pallas import tpu_sc as plsc

NC = 2
NS = 16
NW = NC * NS
CH = 128
NBUF = 4


def _sc_mesh():
    return plsc.VectorSubcoreMesh(
        core_axis_name="c", subcore_axis_name="s",
        num_cores=NC, num_subcores=NS)


def _sc_params():
    return pltpu.CompilerParams(use_tc_tiling_on_sc=False,
                                needs_layout_passes=False)


def _pipeline(nch, body, issue_i, epilogue):
    fg, rem = nch // NBUF, nch % NBUF
    issue_i(0, 0)
    issue_i(1, 1)
    body(0, 0, False, False, True)
    body(1, 1, False, True, True)
    body(2, 2, True, True, True)
    body(3, 3, True, True, True)
    steady_end = fg if rem else fg - 1

    def grp(g, carry):
        for b in range(NBUF):
            body(NBUF * g + b, b, True, True, True)
        return carry

    lax.fori_loop(1, steady_end, grp, 0)
    for c in range(NBUF * steady_end, nch):
        body(c, c % NBUF, True, True, c + 2 < nch)
    epilogue()


def _deg_kernel(npad, e):
    zp = npad // NS
    et = e // NW
    nch = et // CH
    tail = et - nch * CH

    @functools.partial(
        pl.kernel,
        out_type=jax.ShapeDtypeStruct((NC, npad), jnp.float32),
        mesh=_sc_mesh(),
        compiler_params=_sc_params(),
        scratch_types=[
            pltpu.VMEM((NBUF, CH), jnp.int32),
            pltpu.VMEM((16,), jnp.int32),
            pltpu.VMEM((CH,), jnp.float32),
            pltpu.VMEM((zp,), jnp.float32),
            pltpu.VMEM_SHARED((npad,), jnp.float32),
            pltpu.SemaphoreType.DMA((NBUF,)),
            pltpu.SemaphoreType.DMA((2,)),
        ],
    )
    def k(edge_hbm, ones_hbm, zeros_hbm, out_hbm, dsti, dstt, onesb, stage,
          degsh, si, ss):
        cid = lax.axis_index("c")
        sid = lax.axis_index("s")
        wid = sid * NC + cid
        base = wid * et
        pltpu.sync_copy(ones_hbm, onesb)
        pltpu.sync_copy(zeros_hbm, stage)
        pltpu.sync_copy(stage, degsh.at[pl.ds(sid * zp, zp)])
        plsc.subcore_barrier()

        def issue_i(c, b):
            pltpu.async_copy(edge_hbm.at[1, pl.ds(base + c * CH, CH)],
                             dsti.at[b], si.at[b])

        def wait_i(c, b):
            pltpu.make_async_copy(edge_hbm.at[1, pl.ds(base + c * CH, CH)],
                                  dsti.at[b], si.at[b]).wait()

        def issue_s(c, b):
            pltpu.async_copy(onesb, degsh.at[dsti.at[b]], ss.at[b % 2],
                             add=True)

        def wait_s(c, b):
            pltpu.make_async_copy(onesb, degsh.at[dsti.at[b]],
                                  ss.at[b % 2]).wait()

        def body(c, b, w_s2, w_g1, i_i2):
            if w_s2:
                wait_s(c - 2, (b - 2) % NBUF)
            wait_i(c, b)
            issue_s(c, b)
            if i_i2:
                issue_i(c + 2, (b + 2) % NBUF)

        def epilogue():
            wait_s(nch - 2, (nch - 2) % NBUF)
            wait_s(nch - 1, (nch - 1) % NBUF)

        _pipeline(nch, body, issue_i, epilogue)

        if tail:
            pltpu.sync_copy(edge_hbm.at[1, pl.ds(base + nch * CH, tail)], dstt)
            pltpu.sync_copy(onesb.at[pl.ds(0, tail)], degsh.at[dstt], add=True)

        plsc.subcore_barrier()
        pltpu.sync_copy(degsh.at[pl.ds(sid * zp, zp)], stage)
        pltpu.sync_copy(stage, out_hbm.at[cid, pl.ds(sid * zp, zp)])

    return k


def _agg1_kernel(npad, e, h):
    zp = npad // NS
    et = e // NW
    nch = et // CH
    tail = et - nch * CH

    @functools.partial(
        pl.kernel,
        out_type=jax.ShapeDtypeStruct((NC, npad, h), jnp.float32),
        mesh=_sc_mesh(),
        compiler_params=_sc_params(),
        scratch_types=[
            pltpu.VMEM((NBUF, CH), jnp.int32),
            pltpu.VMEM((NBUF, CH), jnp.int32),
            pltpu.VMEM((16,), jnp.int32),
            pltpu.VMEM((16,), jnp.int32),
            pltpu.VMEM((2, CH, h), jnp.float32),
            pltpu.VMEM((16, h), jnp.float32),
            pltpu.VMEM((zp, h), jnp.float32),
            pltpu.VMEM_SHARED((npad, h), jnp.float32),
            pltpu.SemaphoreType.DMA((NBUF,)),
            pltpu.SemaphoreType.DMA((NBUF,)),
            pltpu.SemaphoreType.DMA((2,)),
            pltpu.SemaphoreType.DMA((2,)),
        ],
    )
    def k(g_hbm, edge_hbm, zeros_hbm, out_hbm,
          srci, dsti, srct, dstt, rows, rowst, stage, accsh, sis, sid_, sg, ss):
        cid = lax.axis_index("c")
        sid = lax.axis_index("s")
        wid = sid * NC + cid
        base = wid * et

        @pl.when(cid == 0)
        def _():
            pltpu.sync_copy(g_hbm.at[pl.ds(sid * zp, zp), :], stage)

        @pl.when(cid != 0)
        def _():
            pltpu.sync_copy(zeros_hbm, stage)

        pltpu.sync_copy(stage, accsh.at[pl.ds(sid * zp, zp)])
        plsc.subcore_barrier()

        def issue_i(c, b):
            pltpu.async_copy(edge_hbm.at[0, pl.ds(base + c * CH, CH)],
                             srci.at[b], sis.at[b])
            pltpu.async_copy(edge_hbm.at[1, pl.ds(base + c * CH, CH)],
                             dsti.at[b], sid_.at[b])

        def wait_i(c, b):
            pltpu.make_async_copy(edge_hbm.at[0, pl.ds(base + c * CH, CH)],
                                  srci.at[b], sis.at[b]).wait()
            pltpu.make_async_copy(edge_hbm.at[1, pl.ds(base + c * CH, CH)],
                                  dsti.at[b], sid_.at[b]).wait()

        def issue_g(c, b):
            pltpu.async_copy(g_hbm.at[srci.at[b]], rows.at[b % 2],
                             sg.at[b % 2])

        def wait_g(c, b):
            pltpu.make_async_copy(g_hbm.at[srci.at[b]], rows.at[b % 2],
                                  sg.at[b % 2]).wait()

        def issue_s(c, b):
            pltpu.async_copy(rows.at[b % 2], accsh.at[dsti.at[b]],
                             ss.at[b % 2], add=True)

        def wait_s(c, b):
            pltpu.make_async_copy(rows.at[b % 2], accsh.at[dsti.at[b]],
                                  ss.at[b % 2]).wait()

        def body(c, b, w_s2, w_g1, i_i2):
            if w_s2:
                wait_s(c - 2, (b - 2) % NBUF)
            wait_i(c, b)
            issue_g(c, b)
            if i_i2:
                issue_i(c + 2, (b + 2) % NBUF)
            if w_g1:
                wait_g(c - 1, (b - 1) % NBUF)
                issue_s(c - 1, (b - 1) % NBUF)

        def epilogue():
            wait_g(nch - 1, (nch - 1) % NBUF)
            issue_s(nch - 1, (nch - 1) % NBUF)
            wait_s(nch - 2, (nch - 2) % NBUF)
            wait_s(nch - 1, (nch - 1) % NBUF)

        _pipeline(nch, body, issue_i, epilogue)

        if tail:
            pltpu.sync_copy(edge_hbm.at[0, pl.ds(base + nch * CH, tail)], srct)
            pltpu.sync_copy(edge_hbm.at[1, pl.ds(base + nch * CH, tail)], dstt)
            pltpu.sync_copy(g_hbm.at[srct], rowst)
            pltpu.sync_copy(rowst, accsh.at[dstt], add=True)

        plsc.subcore_barrier()
        pltpu.sync_copy(accsh.at[pl.ds(sid * zp, zp)], stage)
        pltpu.sync_copy(stage, out_hbm.at[cid, pl.ds(sid * zp, zp)])

    return k


def _layer2_kernel(npad, e, h):
    zp = npad // NS
    et = e // NW
    nch = et // CH
    tail = et - nch * CH

    @functools.partial(
        pl.kernel,
        out_type=[
            jax.ShapeDtypeStruct((NC, npad), jnp.float32),
            jax.ShapeDtypeStruct((npad,), jnp.float32),
        ],
        mesh=_sc_mesh(),
        compiler_params=_sc_params(),
        scratch_types=[
            pltpu.VMEM((NBUF, CH), jnp.int32),
            pltpu.VMEM((NBUF, CH), jnp.int32),
            pltpu.VMEM((16,), jnp.int32),
            pltpu.VMEM((16,), jnp.int32),
            pltpu.VMEM((2, CH), jnp.float32),
            pltpu.VMEM((16,), jnp.float32),
            pltpu.VMEM((zp,), jnp.float32),
            pltpu.VMEM((zp * h,), jnp.float32),
            pltpu.VMEM((zp * h,), jnp.float32),
            pltpu.VMEM((zp,), jnp.float32),
            pltpu.VMEM((zp,), jnp.float32),
            pltpu.VMEM((h,), jnp.float32),
            pltpu.VMEM((h,), jnp.float32),
            pltpu.VMEM_SHARED((npad,), jnp.float32),
            pltpu.VMEM_SHARED((npad,), jnp.float32),
            pltpu.SemaphoreType.DMA((NBUF,)),
            pltpu.SemaphoreType.DMA((NBUF,)),
            pltpu.SemaphoreType.DMA((2,)),
            pltpu.SemaphoreType.DMA((2,)),
        ],
    )
    def k(accpf_hbm, dis_hbm, b1_hbm, w2_hbm, edge_hbm, zeros_hbm,
          out_hbm, g2_hbm,
          srci, dsti, srct, dstt, rows, rowst, stage,
          a0, a1, diss, g2b, b1v, w2v, g2sh, accsh,
          sis, sid_, sg, ss):
        cid = lax.axis_index("c")
        sid = lax.axis_index("s")
        wid = sid * NC + cid
        base = wid * et
        r0 = sid * zp

        pltpu.sync_copy(accpf_hbm.at[0, pl.ds(r0 * h, zp * h)], a0)
        pltpu.sync_copy(accpf_hbm.at[1, pl.ds(r0 * h, zp * h)], a1)
        pltpu.sync_copy(dis_hbm.at[pl.ds(r0, zp)], diss)
        pltpu.sync_copy(b1_hbm, b1v)
        pltpu.sync_copy(w2_hbm, w2v)
        pltpu.sync_copy(zeros_hbm, stage)
        pltpu.sync_copy(stage, accsh.at[pl.ds(r0, zp)])
        b1r = b1v[...]
        w2r = w2v[...]
        lanes = lax.iota(jnp.int32, 16)

        def nblk(jb, carry):
            dis16 = diss[pl.ds(jb * 16, 16)]
            g2v = jnp.zeros((16,), jnp.float32)
            for jj in range(16):
                o = (jb * 16 + jj) * h
                arow = a0[pl.ds(o, h)] + a1[pl.ds(o, h)]
                dj = dis16[jj]
                z = jnp.maximum(arow * dj + b1r, 0.0)
                g2v = g2v + jnp.where(lanes == jj, dj * jnp.sum(z * w2r), 0.0)
            g2b[pl.ds(jb * 16, 16)] = g2v
            return carry

        lax.fori_loop(0, zp // 16, nblk, 0)
        pltpu.sync_copy(g2b, g2sh.at[pl.ds(r0, zp)])

        @pl.when(cid == 0)
        def _():
            pltpu.sync_copy(g2b, g2_hbm.at[pl.ds(r0, zp)])

        plsc.subcore_barrier()

        def issue_i(c, b):
            pltpu.async_copy(edge_hbm.at[0, pl.ds(base + c * CH, CH)],
                             srci.at[b], sis.at[b])
            pltpu.async_copy(edge_hbm.at[1, pl.ds(base + c * CH, CH)],
                             dsti.at[b], sid_.at[b])

        def wait_i(c, b):
            pltpu.make_async_copy(edge_hbm.at[0, pl.ds(base + c * CH, CH)],
                                  srci.at[b], sis.at[b]).wait()
            pltpu.make_async_copy(edge_hbm.at[1, pl.ds(base + c * CH, CH)],
                                  dsti.at[b], sid_.at[b]).wait()

        def issue_g(c, b):
            pltpu.async_copy(g2sh.at[srci.at[b]], rows.at[b % 2],
                             sg.at[b % 2])

        def wait_g(c, b):
            pltpu.make_async_copy(g2sh.at[srci.at[b]], rows.at[b % 2],
                                  sg.at[b % 2]).wait()

        def issue_s(c, b):
            pltpu.async_copy(rows.at[b % 2], accsh.at[dsti.at[b]],
                             ss.at[b % 2], add=True)

        def wait_s(c, b):
            pltpu.make_async_copy(rows.at[b % 2], accsh.at[dsti.at[b]],
                                  ss.at[b % 2]).wait()

        def body(c, b, w_s2, w_g1, i_i2):
            if w_s2:
                wait_s(c - 2, (b - 2) % NBUF)
            wait_i(c, b)
            issue_g(c, b)
            if i_i2:
                issue_i(c + 2, (b + 2) % NBUF)
            if w_g1:
                wait_g(c - 1, (b - 1) % NBUF)
                issue_s(c - 1, (b - 1) % NBUF)

        def epilogue():
            wait_g(nch - 1, (nch - 1) % NBUF)
            issue_s(nch - 1, (nch - 1) % NBUF)
            wait_s(nch - 2, (nch - 2) % NBUF)
            wait_s(nch - 1, (nch - 1) % NBUF)

        _pipeline(nch, body, issue_i, epilogue)

        if tail:
            pltpu.sync_copy(edge_hbm.at[0, pl.ds(base + nch * CH, tail)], srct)
            pltpu.sync_copy(edge_hbm.at[1, pl.ds(base + nch * CH, tail)], dstt)
            pltpu.sync_copy(g2sh.at[srct], rowst)
            pltpu.sync_copy(rowst, accsh.at[dstt], add=True)

        plsc.subcore_barrier()
        pltpu.sync_copy(accsh.at[pl.ds(r0, zp)], stage)
        pltpu.sync_copy(stage, out_hbm.at[cid, pl.ds(r0, zp)])

    return k


def _tc1_body(degp_ref, x_ref, w1_ref, g1_ref, dis_ref):
    deg = degp_ref[0, :] + degp_ref[1, :] + 1.0
    dis = lax.rsqrt(deg)
    hm = jnp.dot(x_ref[...], w1_ref[...], preferred_element_type=jnp.float32)
    g1_ref[...] = hm * dis[:, None]
    dis_ref[...] = dis


def _tc3_body(aggp_ref, g2_ref, dis_ref, b2_ref, out_ref):
    s = aggp_ref[0] + aggp_ref[1] + g2_ref[...]
    out_ref[...] = jax.nn.sigmoid(dis_ref[...] * s + b2_ref[0, 0])[:, None]


def kernel(x, edge_index, W1, b1, W2, b2):
    n, d = x.shape
    h = W1.shape[1]
    e = edge_index.shape[1]

    blk = 1024
    npad = ((n + 1 + blk - 1) // blk) * blk
    grid = npad // blk
    zp = npad // NS

    xp = jnp.pad(x, ((0, npad - n), (0, 0)))
    ones_ch = jnp.ones((CH,), jnp.float32)
    zeros1 = jnp.zeros((zp,), jnp.float32)
    zeros2 = jnp.zeros((zp, h), jnp.float32)

    degp = _deg_kernel(npad, e)(edge_index, ones_ch, zeros1)

    g1, dis = pl.pallas_call(
        _tc1_body,
        grid=(grid,),
        in_specs=[
            pl.BlockSpec((NC, blk), lambda i: (0, i)),
            pl.BlockSpec((blk, d), lambda i: (i, 0)),
            pl.BlockSpec((d, h), lambda i: (0, 0)),
        ],
        out_specs=[
            pl.BlockSpec((blk, h), lambda i: (i, 0)),
            pl.BlockSpec((blk,), lambda i: (i,)),
        ],
        out_shape=[
            jax.ShapeDtypeStruct((npad, h), jnp.float32),
            jax.ShapeDtypeStruct((npad,), jnp.float32),
        ],
    )(degp, xp, W1)

    accp = _agg1_kernel(npad, e, h)(g1, edge_index, zeros2)

    agg2p, g2 = _layer2_kernel(npad, e, h)(
        accp.reshape(NC, npad * h), dis, b1, W2.reshape(h), edge_index, zeros1)

    out = pl.pallas_call(
        _tc3_body,
        grid=(grid,),
        in_specs=[
            pl.BlockSpec((NC, blk), lambda i: (0, i)),
            pl.BlockSpec((blk,), lambda i: (i,)),
            pl.BlockSpec((blk,), lambda i: (i,)),
            pl.BlockSpec((1, 1), lambda i: (0, 0)),
        ],
        out_specs=pl.BlockSpec((blk, 1), lambda i: (i, 0)),
        out_shape=jax.ShapeDtypeStruct((npad, 1), jnp.float32),
    )(agg2p, g2, dis, b2.reshape(1, 1))

    return out[:n, :]


# speedup vs baseline: 79.1479x; 1.1811x over previous
import functools

import jax
import jax.numpy as jnp
from jax import lax
from jax.experimental import pallas as pl
from jax.experimental.pallas import tpu as pltpu
from jax.experimental.pallas import tpu_sc as plsc

NC = 2
NS = 16
NW = NC * NS
CH = 128
NBUF = 4


def _sc_mesh():
    return plsc.VectorSubcoreMesh(
        core_axis_name="c", subcore_axis_name="s",
        num_cores=NC, num_subcores=NS)


def _sc_params():
    return pltpu.CompilerParams(use_tc_tiling_on_sc=False,
                                needs_layout_passes=False)


def _fill(ref, size, value):
    v = jnp.full((16,), value, jnp.float32)

    def st(i, carry):
        ref[pl.ds(i * 16, 16)] = v
        return carry

    lax.fori_loop(0, size // 16, st, 0)


def _pipeline(nch, body, issue_i, epilogue):
    fg, rem = nch // NBUF, nch % NBUF
    issue_i(0, 0)
    issue_i(1, 1)
    body(0, 0, False, False, True)
    body(1, 1, False, True, True)
    body(2, 2, True, True, True)
    body(3, 3, True, True, True)
    steady_end = fg if rem else fg - 1

    def grp(g, carry):
        for b in range(NBUF):
            body(NBUF * g + b, b, True, True, True)
        return carry

    lax.fori_loop(1, steady_end, grp, 0)
    for c in range(NBUF * steady_end, nch):
        body(c, c % NBUF, True, True, c + 2 < nch)
    epilogue()


def _deg_kernel(npad, e):
    zp = npad // NS
    et = e // NW
    nch = et // CH
    tail = et - nch * CH

    @functools.partial(
        pl.kernel,
        out_type=jax.ShapeDtypeStruct((NC, npad), jnp.float32),
        mesh=_sc_mesh(),
        compiler_params=_sc_params(),
        scratch_types=[
            pltpu.VMEM((NBUF, CH), jnp.int32),
            pltpu.VMEM((16,), jnp.int32),
            pltpu.VMEM((CH,), jnp.float32),
            pltpu.VMEM((zp,), jnp.float32),
            pltpu.VMEM_SHARED((npad,), jnp.float32),
            pltpu.SemaphoreType.DMA((NBUF,)),
            pltpu.SemaphoreType.DMA((2,)),
        ],
    )
    def k(edge_hbm, out_hbm, dsti, dstt, onesb, stage, degsh, si, ss):
        cid = lax.axis_index("c")
        sid = lax.axis_index("s")
        wid = sid * NC + cid
        base = wid * et
        _fill(onesb, CH, 1.0)
        _fill(stage, zp, 0.0)
        pltpu.sync_copy(stage, degsh.at[pl.ds(sid * zp, zp)])
        plsc.subcore_barrier()

        def issue_i(c, b):
            pltpu.async_copy(edge_hbm.at[1, pl.ds(base + c * CH, CH)],
                             dsti.at[b], si.at[b])

        def wait_i(c, b):
            pltpu.make_async_copy(edge_hbm.at[1, pl.ds(base + c * CH, CH)],
                                  dsti.at[b], si.at[b]).wait()

        def issue_s(c, b):
            pltpu.async_copy(onesb, degsh.at[dsti.at[b]], ss.at[b % 2],
                             add=True)

        def wait_s(c, b):
            pltpu.make_async_copy(onesb, degsh.at[dsti.at[b]],
                                  ss.at[b % 2]).wait()

        def body(c, b, w_s2, w_g1, i_i2):
            if w_s2:
                wait_s(c - 2, (b - 2) % NBUF)
            wait_i(c, b)
            issue_s(c, b)
            if i_i2:
                issue_i(c + 2, (b + 2) % NBUF)

        def epilogue():
            wait_s(nch - 2, (nch - 2) % NBUF)
            wait_s(nch - 1, (nch - 1) % NBUF)

        _pipeline(nch, body, issue_i, epilogue)

        if tail:
            pltpu.sync_copy(edge_hbm.at[1, pl.ds(base + nch * CH, tail)], dstt)
            pltpu.sync_copy(onesb.at[pl.ds(0, tail)], degsh.at[dstt], add=True)

        plsc.subcore_barrier()
        pltpu.sync_copy(degsh.at[pl.ds(sid * zp, zp)], stage)
        pltpu.sync_copy(stage, out_hbm.at[cid, pl.ds(sid * zp, zp)])

    return k


def _agg1_kernel(npad, e, h):
    zp = npad // NS
    et = e // NW
    nch = et // CH
    tail = et - nch * CH

    @functools.partial(
        pl.kernel,
        out_type=jax.ShapeDtypeStruct((NC, npad, h), jnp.float32),
        mesh=_sc_mesh(),
        compiler_params=_sc_params(),
        scratch_types=[
            pltpu.VMEM((NBUF, CH), jnp.int32),
            pltpu.VMEM((NBUF, CH), jnp.int32),
            pltpu.VMEM((16,), jnp.int32),
            pltpu.VMEM((16,), jnp.int32),
            pltpu.VMEM((2, CH, h), jnp.float32),
            pltpu.VMEM((16, h), jnp.float32),
            pltpu.VMEM((zp, h), jnp.float32),
            pltpu.VMEM((zp * h,), jnp.float32),
            pltpu.VMEM((16, h), jnp.float32),
            pltpu.VMEM((16, h), jnp.float32),
            pltpu.VMEM((zp,), jnp.float32),
            pltpu.VMEM_SHARED((npad, h), jnp.float32),
            pltpu.VMEM_SHARED((npad, h), jnp.float32),
            pltpu.SemaphoreType.DMA((NBUF,)),
            pltpu.SemaphoreType.DMA((NBUF,)),
            pltpu.SemaphoreType.DMA((2,)),
            pltpu.SemaphoreType.DMA((2,)),
        ],
    )
    def k(hf_hbm, dis_hbm, edge_hbm, out_hbm,
          srci, dsti, srct, dstt, rows, rowst, stage, hbuf, tb, ztb, diss,
          g1sh, accsh, sis, sid_, sg, ss):
        cid = lax.axis_index("c")
        sid = lax.axis_index("s")
        wid = sid * NC + cid
        base = wid * et
        r0 = sid * zp

        pltpu.sync_copy(hf_hbm.at[pl.ds(r0 * h, zp * h)], hbuf)
        pltpu.sync_copy(dis_hbm.at[pl.ds(r0, zp)], diss)
        zrow = jnp.zeros((h,), jnp.float32)
        for jj in range(16):
            ztb[jj, :] = zrow

        def nblk(jb, carry):
            dis16 = diss[pl.ds(jb * 16, 16)]
            for jj in range(16):
                o = (jb * 16 + jj) * h
                tb[jj, :] = hbuf[pl.ds(o, h)] * dis16[jj]
            rows16 = pl.ds(r0 + jb * 16, 16)
            pltpu.sync_copy(tb, g1sh.at[rows16, :])

            @pl.when(cid == 0)
            def _():
                pltpu.sync_copy(tb, accsh.at[rows16, :])

            @pl.when(cid != 0)
            def _():
                pltpu.sync_copy(ztb, accsh.at[rows16, :])

            return carry

        lax.fori_loop(0, zp // 16, nblk, 0)
        plsc.subcore_barrier()

        def issue_i(c, b):
            pltpu.async_copy(edge_hbm.at[0, pl.ds(base + c * CH, CH)],
                             srci.at[b], sis.at[b])
            pltpu.async_copy(edge_hbm.at[1, pl.ds(base + c * CH, CH)],
                             dsti.at[b], sid_.at[b])

        def wait_i(c, b):
            pltpu.make_async_copy(edge_hbm.at[0, pl.ds(base + c * CH, CH)],
                                  srci.at[b], sis.at[b]).wait()
            pltpu.make_async_copy(edge_hbm.at[1, pl.ds(base + c * CH, CH)],
                                  dsti.at[b], sid_.at[b]).wait()

        def issue_g(c, b):
            pltpu.async_copy(g1sh.at[srci.at[b]], rows.at[b % 2],
                             sg.at[b % 2])

        def wait_g(c, b):
            pltpu.make_async_copy(g1sh.at[srci.at[b]], rows.at[b % 2],
                                  sg.at[b % 2]).wait()

        def issue_s(c, b):
            pltpu.async_copy(rows.at[b % 2], accsh.at[dsti.at[b]],
                             ss.at[b % 2], add=True)

        def wait_s(c, b):
            pltpu.make_async_copy(rows.at[b % 2], accsh.at[dsti.at[b]],
                                  ss.at[b % 2]).wait()

        def body(c, b, w_s2, w_g1, i_i2):
            if w_s2:
                wait_s(c - 2, (b - 2) % NBUF)
            wait_i(c, b)
            issue_g(c, b)
            if i_i2:
                issue_i(c + 2, (b + 2) % NBUF)
            if w_g1:
                wait_g(c - 1, (b - 1) % NBUF)
                issue_s(c - 1, (b - 1) % NBUF)

        def epilogue():
            wait_g(nch - 1, (nch - 1) % NBUF)
            issue_s(nch - 1, (nch - 1) % NBUF)
            wait_s(nch - 2, (nch - 2) % NBUF)
            wait_s(nch - 1, (nch - 1) % NBUF)

        _pipeline(nch, body, issue_i, epilogue)

        if tail:
            pltpu.sync_copy(edge_hbm.at[0, pl.ds(base + nch * CH, tail)], srct)
            pltpu.sync_copy(edge_hbm.at[1, pl.ds(base + nch * CH, tail)], dstt)
            pltpu.sync_copy(g1sh.at[srct], rowst)
            pltpu.sync_copy(rowst, accsh.at[dstt], add=True)

        plsc.subcore_barrier()
        pltpu.sync_copy(accsh.at[pl.ds(r0, zp), :], stage)
        pltpu.sync_copy(stage, out_hbm.at[cid, pl.ds(r0, zp)])

    return k


def _layer2_kernel(npad, e, h):
    zp = npad // NS
    et = e // NW
    nch = et // CH
    tail = et - nch * CH

    @functools.partial(
        pl.kernel,
        out_type=[
            jax.ShapeDtypeStruct((NC, npad), jnp.float32),
            jax.ShapeDtypeStruct((npad,), jnp.float32),
        ],
        mesh=_sc_mesh(),
        compiler_params=_sc_params(),
        scratch_types=[
            pltpu.VMEM((NBUF, CH), jnp.int32),
            pltpu.VMEM((NBUF, CH), jnp.int32),
            pltpu.VMEM((16,), jnp.int32),
            pltpu.VMEM((16,), jnp.int32),
            pltpu.VMEM((2, CH), jnp.float32),
            pltpu.VMEM((16,), jnp.float32),
            pltpu.VMEM((zp,), jnp.float32),
            pltpu.VMEM((zp * h,), jnp.float32),
            pltpu.VMEM((zp * h,), jnp.float32),
            pltpu.VMEM((zp,), jnp.float32),
            pltpu.VMEM((zp,), jnp.float32),
            pltpu.VMEM((h,), jnp.float32),
            pltpu.VMEM((h,), jnp.float32),
            pltpu.VMEM_SHARED((npad,), jnp.float32),
            pltpu.VMEM_SHARED((npad,), jnp.float32),
            pltpu.SemaphoreType.DMA((NBUF,)),
            pltpu.SemaphoreType.DMA((NBUF,)),
            pltpu.SemaphoreType.DMA((2,)),
            pltpu.SemaphoreType.DMA((2,)),
        ],
    )
    def k(accpf_hbm, dis_hbm, b1_hbm, w2_hbm, edge_hbm,
          out_hbm, g2_hbm,
          srci, dsti, srct, dstt, rows, rowst, stage,
          a0, a1, diss, g2b, b1v, w2v, g2sh, accsh,
          sis, sid_, sg, ss):
        cid = lax.axis_index("c")
        sid = lax.axis_index("s")
        wid = sid * NC + cid
        base = wid * et
        r0 = sid * zp

        pltpu.sync_copy(accpf_hbm.at[0, pl.ds(r0 * h, zp * h)], a0)
        pltpu.sync_copy(accpf_hbm.at[1, pl.ds(r0 * h, zp * h)], a1)
        pltpu.sync_copy(dis_hbm.at[pl.ds(r0, zp)], diss)
        pltpu.sync_copy(b1_hbm, b1v)
        pltpu.sync_copy(w2_hbm, w2v)
        _fill(stage, zp, 0.0)
        pltpu.sync_copy(stage, accsh.at[pl.ds(r0, zp)])
        b1r = b1v[...]
        w2r = w2v[...]
        lanes = lax.iota(jnp.int32, 16)

        def nblk(jb, carry):
            dis16 = diss[pl.ds(jb * 16, 16)]
            g2v = jnp.zeros((16,), jnp.float32)
            for jj in range(16):
                o = (jb * 16 + jj) * h
                arow = a0[pl.ds(o, h)] + a1[pl.ds(o, h)]
                dj = dis16[jj]
                z = jnp.maximum(arow * dj + b1r, 0.0)
                g2v = g2v + jnp.where(lanes == jj, dj * jnp.sum(z * w2r), 0.0)
            g2b[pl.ds(jb * 16, 16)] = g2v
            return carry

        lax.fori_loop(0, zp // 16, nblk, 0)
        pltpu.sync_copy(g2b, g2sh.at[pl.ds(r0, zp)])

        @pl.when(cid == 0)
        def _():
            pltpu.sync_copy(g2b, g2_hbm.at[pl.ds(r0, zp)])

        plsc.subcore_barrier()

        def issue_i(c, b):
            pltpu.async_copy(edge_hbm.at[0, pl.ds(base + c * CH, CH)],
                             srci.at[b], sis.at[b])
            pltpu.async_copy(edge_hbm.at[1, pl.ds(base + c * CH, CH)],
                             dsti.at[b], sid_.at[b])

        def wait_i(c, b):
            pltpu.make_async_copy(edge_hbm.at[0, pl.ds(base + c * CH, CH)],
                                  srci.at[b], sis.at[b]).wait()
            pltpu.make_async_copy(edge_hbm.at[1, pl.ds(base + c * CH, CH)],
                                  dsti.at[b], sid_.at[b]).wait()

        def issue_g(c, b):
            pltpu.async_copy(g2sh.at[srci.at[b]], rows.at[b % 2],
                             sg.at[b % 2])

        def wait_g(c, b):
            pltpu.make_async_copy(g2sh.at[srci.at[b]], rows.at[b % 2],
                                  sg.at[b % 2]).wait()

        def issue_s(c, b):
            pltpu.async_copy(rows.at[b % 2], accsh.at[dsti.at[b]],
                             ss.at[b % 2], add=True)

        def wait_s(c, b):
            pltpu.make_async_copy(rows.at[b % 2], accsh.at[dsti.at[b]],
                                  ss.at[b % 2]).wait()

        def body(c, b, w_s2, w_g1, i_i2):
            if w_s2:
                wait_s(c - 2, (b - 2) % NBUF)
            wait_i(c, b)
            issue_g(c, b)
            if i_i2:
                issue_i(c + 2, (b + 2) % NBUF)
            if w_g1:
                wait_g(c - 1, (b - 1) % NBUF)
                issue_s(c - 1, (b - 1) % NBUF)

        def epilogue():
            wait_g(nch - 1, (nch - 1) % NBUF)
            issue_s(nch - 1, (nch - 1) % NBUF)
            wait_s(nch - 2, (nch - 2) % NBUF)
            wait_s(nch - 1, (nch - 1) % NBUF)

        _pipeline(nch, body, issue_i, epilogue)

        if tail:
            pltpu.sync_copy(edge_hbm.at[0, pl.ds(base + nch * CH, tail)], srct)
            pltpu.sync_copy(edge_hbm.at[1, pl.ds(base + nch * CH, tail)], dstt)
            pltpu.sync_copy(g2sh.at[srct], rowst)
            pltpu.sync_copy(rowst, accsh.at[dstt], add=True)

        plsc.subcore_barrier()
        pltpu.sync_copy(accsh.at[pl.ds(r0, zp)], stage)
        pltpu.sync_copy(stage, out_hbm.at[cid, pl.ds(r0, zp)])

    return k


def _mm_body(x_ref, w1_ref, hm_ref):
    hm_ref[...] = jnp.dot(x_ref[...], w1_ref[...],
                          preferred_element_type=jnp.float32)


def _dis_body(degp_ref, dis_ref):
    dis_ref[...] = lax.rsqrt(degp_ref[0, :] + degp_ref[1, :] + 1.0)


def _tc3_body(aggp_ref, g2_ref, dis_ref, b2_ref, out_ref):
    s = aggp_ref[0] + aggp_ref[1] + g2_ref[...]
    out_ref[...] = jax.nn.sigmoid(dis_ref[...] * s + b2_ref[0, 0])[:, None]


def kernel(x, edge_index, W1, b1, W2, b2):
    n, d = x.shape
    h = W1.shape[1]
    e = edge_index.shape[1]

    blk = 1024
    npad = ((n + 1 + blk - 1) // blk) * blk
    grid = npad // blk

    xp = jnp.pad(x, ((0, npad - n), (0, 0)))

    degp = _deg_kernel(npad, e)(edge_index)

    hm = pl.pallas_call(
        _mm_body,
        grid=(grid,),
        in_specs=[
            pl.BlockSpec((blk, d), lambda i: (i, 0)),
            pl.BlockSpec((d, h), lambda i: (0, 0)),
        ],
        out_specs=pl.BlockSpec((blk, h), lambda i: (i, 0)),
        out_shape=jax.ShapeDtypeStruct((npad, h), jnp.float32),
    )(xp, W1)

    dis = pl.pallas_call(
        _dis_body,
        grid=(grid,),
        in_specs=[pl.BlockSpec((NC, blk), lambda i: (0, i))],
        out_specs=pl.BlockSpec((blk,), lambda i: (i,)),
        out_shape=jax.ShapeDtypeStruct((npad,), jnp.float32),
    )(degp)

    accp = _agg1_kernel(npad, e, h)(hm.reshape(npad * h), dis, edge_index)

    agg2p, g2 = _layer2_kernel(npad, e, h)(
        accp.reshape(NC, npad * h), dis, b1, W2.reshape(h), edge_index)

    out = pl.pallas_call(
        _tc3_body,
        grid=(grid,),
        in_specs=[
            pl.BlockSpec((NC, blk), lambda i: (0, i)),
            pl.BlockSpec((blk,), lambda i: (i,)),
            pl.BlockSpec((blk,), lambda i: (i,)),
            pl.BlockSpec((1, 1), lambda i: (0, 0)),
        ],
        out_specs=pl.BlockSpec((blk, 1), lambda i: (i, 0)),
        out_shape=jax.ShapeDtypeStruct((n, 1), jnp.float32),
    )(agg2p, g2, dis, b2.reshape(1, 1))

    return out


# speedup vs baseline: 81.7432x; 1.0328x over previous
import functools

import jax
import jax.numpy as jnp
from jax import lax
from jax.experimental import pallas as pl
from jax.experimental.pallas import tpu as pltpu
from jax.experimental.pallas import tpu_sc as plsc

NC = 2
NS = 16
NW = NC * NS
CH = 128
NBUF = 4


def _sc_mesh():
    return plsc.VectorSubcoreMesh(
        core_axis_name="c", subcore_axis_name="s",
        num_cores=NC, num_subcores=NS)


def _sc_params(tc_tiling=False):
    return pltpu.CompilerParams(use_tc_tiling_on_sc=tc_tiling,
                                needs_layout_passes=False)


def _rsqrt16(x):
    yi = jnp.int32(0x5F3759DF) - (plsc.bitcast(x, jnp.int32) >> 1)
    y = plsc.bitcast(yi, jnp.float32)
    for _ in range(4):
        y = y * (1.5 - 0.5 * x * y * y)
    return y


def _fill(ref, size, value):
    v = jnp.full((16,), value, jnp.float32)

    def st(i, carry):
        ref[pl.ds(i * 16, 16)] = v
        return carry

    lax.fori_loop(0, size // 16, st, 0)


def _pipeline(nch, body, issue_i, epilogue):
    fg, rem = nch // NBUF, nch % NBUF
    issue_i(0, 0)
    issue_i(1, 1)
    body(0, 0, False, False, True)
    body(1, 1, False, True, True)
    body(2, 2, True, True, True)
    body(3, 3, True, True, True)
    steady_end = fg if rem else fg - 1

    def grp(g, carry):
        for b in range(NBUF):
            body(NBUF * g + b, b, True, True, True)
        return carry

    lax.fori_loop(1, steady_end, grp, 0)
    for c in range(NBUF * steady_end, nch):
        body(c, c % NBUF, True, True, c + 2 < nch)
    epilogue()


def _deg_kernel(npad, e):
    zp = npad // NS
    et = e // NW
    nch = et // CH
    tail = et - nch * CH

    @functools.partial(
        pl.kernel,
        out_type=jax.ShapeDtypeStruct((NC, npad), jnp.float32),
        mesh=_sc_mesh(),
        compiler_params=_sc_params(),
        scratch_types=[
            pltpu.VMEM((NBUF, CH), jnp.int32),
            pltpu.VMEM((16,), jnp.int32),
            pltpu.VMEM((CH,), jnp.float32),
            pltpu.VMEM((zp,), jnp.float32),
            pltpu.VMEM_SHARED((npad,), jnp.float32),
            pltpu.SemaphoreType.DMA((NBUF,)),
            pltpu.SemaphoreType.DMA((2,)),
        ],
    )
    def k(edge_hbm, out_hbm, dsti, dstt, onesb, stage, degsh, si, ss):
        cid = lax.axis_index("c")
        sid = lax.axis_index("s")
        wid = sid * NC + cid
        base = wid * et
        _fill(onesb, CH, 1.0)
        _fill(stage, zp, 0.0)
        pltpu.sync_copy(stage, degsh.at[pl.ds(sid * zp, zp)])
        plsc.subcore_barrier()

        def issue_i(c, b):
            pltpu.async_copy(edge_hbm.at[1, pl.ds(base + c * CH, CH)],
                             dsti.at[b], si.at[b])

        def wait_i(c, b):
            pltpu.make_async_copy(edge_hbm.at[1, pl.ds(base + c * CH, CH)],
                                  dsti.at[b], si.at[b]).wait()

        def issue_s(c, b):
            pltpu.async_copy(onesb, degsh.at[dsti.at[b]], ss.at[b % 2],
                             add=True)

        def wait_s(c, b):
            pltpu.make_async_copy(onesb, degsh.at[dsti.at[b]],
                                  ss.at[b % 2]).wait()

        def body(c, b, w_s2, w_g1, i_i2):
            if w_s2:
                wait_s(c - 2, (b - 2) % NBUF)
            wait_i(c, b)
            issue_s(c, b)
            if i_i2:
                issue_i(c + 2, (b + 2) % NBUF)

        def epilogue():
            wait_s(nch - 2, (nch - 2) % NBUF)
            wait_s(nch - 1, (nch - 1) % NBUF)

        _pipeline(nch, body, issue_i, epilogue)

        if tail:
            pltpu.sync_copy(edge_hbm.at[1, pl.ds(base + nch * CH, tail)], dstt)
            pltpu.sync_copy(onesb.at[pl.ds(0, tail)], degsh.at[dstt], add=True)

        plsc.subcore_barrier()
        pltpu.sync_copy(degsh.at[pl.ds(sid * zp, zp)], stage)
        pltpu.sync_copy(stage, out_hbm.at[cid, pl.ds(sid * zp, zp)])

    return k


def _agg1_kernel(npad, e, h):
    zp = npad // NS
    et = e // NW
    nch = et // CH
    tail = et - nch * CH

    @functools.partial(
        pl.kernel,
        out_type=[
            jax.ShapeDtypeStruct((NC, npad, h), jnp.float32),
            jax.ShapeDtypeStruct((npad,), jnp.float32),
        ],
        mesh=_sc_mesh(),
        compiler_params=_sc_params(),
        scratch_types=[
            pltpu.VMEM((NBUF, CH), jnp.int32),
            pltpu.VMEM((NBUF, CH), jnp.int32),
            pltpu.VMEM((16,), jnp.int32),
            pltpu.VMEM((16,), jnp.int32),
            pltpu.VMEM((2, CH, h), jnp.float32),
            pltpu.VMEM((16, h), jnp.float32),
            pltpu.VMEM((zp, h), jnp.float32),
            pltpu.VMEM((zp * h,), jnp.float32),
            pltpu.VMEM((16, h), jnp.float32),
            pltpu.VMEM((16, h), jnp.float32),
            pltpu.VMEM((zp,), jnp.float32),
            pltpu.VMEM((zp,), jnp.float32),
            pltpu.VMEM((zp,), jnp.float32),
            pltpu.VMEM_SHARED((npad, h), jnp.float32),
            pltpu.VMEM_SHARED((npad, h), jnp.float32),
            pltpu.SemaphoreType.DMA((NBUF,)),
            pltpu.SemaphoreType.DMA((NBUF,)),
            pltpu.SemaphoreType.DMA((2,)),
            pltpu.SemaphoreType.DMA((2,)),
        ],
    )
    def k(hf_hbm, degp_hbm, edge_hbm, out_hbm, dis_hbm,
          srci, dsti, srct, dstt, rows, rowst, stage, hbuf, tb, ztb,
          d0b, d1b, diss, g1sh, accsh, sis, sid_, sg, ss):
        cid = lax.axis_index("c")
        sid = lax.axis_index("s")
        wid = sid * NC + cid
        base = wid * et
        r0 = sid * zp

        pltpu.sync_copy(hf_hbm.at[pl.ds(r0 * h, zp * h)], hbuf)
        pltpu.sync_copy(degp_hbm.at[0, pl.ds(r0, zp)], d0b)
        pltpu.sync_copy(degp_hbm.at[1, pl.ds(r0, zp)], d1b)
        zrow = jnp.zeros((h,), jnp.float32)
        for jj in range(16):
            ztb[jj, :] = zrow

        def nblk(jb, carry):
            j16 = pl.ds(jb * 16, 16)
            dis16 = _rsqrt16(d0b[j16] + d1b[j16] + 1.0)
            diss[j16] = dis16
            for jj in range(16):
                o = (jb * 16 + jj) * h
                tb[jj, :] = hbuf[pl.ds(o, h)] * dis16[jj]
            rows16 = pl.ds(r0 + jb * 16, 16)
            pltpu.sync_copy(tb, g1sh.at[rows16, :])

            @pl.when(cid == 0)
            def _():
                pltpu.sync_copy(tb, accsh.at[rows16, :])

            @pl.when(cid != 0)
            def _():
                pltpu.sync_copy(ztb, accsh.at[rows16, :])

            return carry

        lax.fori_loop(0, zp // 16, nblk, 0)

        @pl.when(cid == 0)
        def _():
            pltpu.sync_copy(diss, dis_hbm.at[pl.ds(r0, zp)])

        plsc.subcore_barrier()

        def issue_i(c, b):
            pltpu.async_copy(edge_hbm.at[0, pl.ds(base + c * CH, CH)],
                             srci.at[b], sis.at[b])
            pltpu.async_copy(edge_hbm.at[1, pl.ds(base + c * CH, CH)],
                             dsti.at[b], sid_.at[b])

        def wait_i(c, b):
            pltpu.make_async_copy(edge_hbm.at[0, pl.ds(base + c * CH, CH)],
                                  srci.at[b], sis.at[b]).wait()
            pltpu.make_async_copy(edge_hbm.at[1, pl.ds(base + c * CH, CH)],
                                  dsti.at[b], sid_.at[b]).wait()

        def issue_g(c, b):
            pltpu.async_copy(g1sh.at[srci.at[b]], rows.at[b % 2],
                             sg.at[b % 2])

        def wait_g(c, b):
            pltpu.make_async_copy(g1sh.at[srci.at[b]], rows.at[b % 2],
                                  sg.at[b % 2]).wait()

        def issue_s(c, b):
            pltpu.async_copy(rows.at[b % 2], accsh.at[dsti.at[b]],
                             ss.at[b % 2], add=True)

        def wait_s(c, b):
            pltpu.make_async_copy(rows.at[b % 2], accsh.at[dsti.at[b]],
                                  ss.at[b % 2]).wait()

        def body(c, b, w_s2, w_g1, i_i2):
            if w_s2:
                wait_s(c - 2, (b - 2) % NBUF)
            wait_i(c, b)
            issue_g(c, b)
            if i_i2:
                issue_i(c + 2, (b + 2) % NBUF)
            if w_g1:
                wait_g(c - 1, (b - 1) % NBUF)
                issue_s(c - 1, (b - 1) % NBUF)

        def epilogue():
            wait_g(nch - 1, (nch - 1) % NBUF)
            issue_s(nch - 1, (nch - 1) % NBUF)
            wait_s(nch - 2, (nch - 2) % NBUF)
            wait_s(nch - 1, (nch - 1) % NBUF)

        _pipeline(nch, body, issue_i, epilogue)

        if tail:
            pltpu.sync_copy(edge_hbm.at[0, pl.ds(base + nch * CH, tail)], srct)
            pltpu.sync_copy(edge_hbm.at[1, pl.ds(base + nch * CH, tail)], dstt)
            pltpu.sync_copy(g1sh.at[srct], rowst)
            pltpu.sync_copy(rowst, accsh.at[dstt], add=True)

        plsc.subcore_barrier()
        pltpu.sync_copy(accsh.at[pl.ds(r0, zp), :], stage)
        pltpu.sync_copy(stage, out_hbm.at[cid, pl.ds(r0, zp)])

    return k


def _layer2_kernel(npad, e, h):
    zp = npad // NS
    et = e // NW
    nch = et // CH
    tail = et - nch * CH

    @functools.partial(
        pl.kernel,
        out_type=[
            jax.ShapeDtypeStruct((NC, npad), jnp.float32),
            jax.ShapeDtypeStruct((npad,), jnp.float32),
        ],
        mesh=_sc_mesh(),
        compiler_params=_sc_params(),
        scratch_types=[
            pltpu.VMEM((NBUF, CH), jnp.int32),
            pltpu.VMEM((NBUF, CH), jnp.int32),
            pltpu.VMEM((16,), jnp.int32),
            pltpu.VMEM((16,), jnp.int32),
            pltpu.VMEM((2, CH), jnp.float32),
            pltpu.VMEM((16,), jnp.float32),
            pltpu.VMEM((zp,), jnp.float32),
            pltpu.VMEM((zp * h,), jnp.float32),
            pltpu.VMEM((zp * h,), jnp.float32),
            pltpu.VMEM((zp,), jnp.float32),
            pltpu.VMEM((zp,), jnp.float32),
            pltpu.VMEM((h,), jnp.float32),
            pltpu.VMEM((h,), jnp.float32),
            pltpu.VMEM_SHARED((npad,), jnp.float32),
            pltpu.VMEM_SHARED((npad,), jnp.float32),
            pltpu.SemaphoreType.DMA((NBUF,)),
            pltpu.SemaphoreType.DMA((NBUF,)),
            pltpu.SemaphoreType.DMA((2,)),
            pltpu.SemaphoreType.DMA((2,)),
        ],
    )
    def k(accpf_hbm, dis_hbm, b1_hbm, w2_hbm, edge_hbm,
          out_hbm, g2_hbm,
          srci, dsti, srct, dstt, rows, rowst, stage,
          a0, a1, diss, g2b, b1v, w2v, g2sh, accsh,
          sis, sid_, sg, ss):
        cid = lax.axis_index("c")
        sid = lax.axis_index("s")
        wid = sid * NC + cid
        base = wid * et
        r0 = sid * zp

        pltpu.sync_copy(accpf_hbm.at[0, pl.ds(r0 * h, zp * h)], a0)
        pltpu.sync_copy(accpf_hbm.at[1, pl.ds(r0 * h, zp * h)], a1)
        pltpu.sync_copy(dis_hbm.at[pl.ds(r0, zp)], diss)
        pltpu.sync_copy(b1_hbm, b1v)
        pltpu.sync_copy(w2_hbm, w2v)
        _fill(stage, zp, 0.0)
        pltpu.sync_copy(stage, accsh.at[pl.ds(r0, zp)])
        b1r = b1v[...]
        w2r = w2v[...]
        lanes = lax.iota(jnp.int32, 16)

        def nblk(jb, carry):
            dis16 = diss[pl.ds(jb * 16, 16)]
            g2v = jnp.zeros((16,), jnp.float32)
            for jj in range(16):
                o = (jb * 16 + jj) * h
                arow = a0[pl.ds(o, h)] + a1[pl.ds(o, h)]
                dj = dis16[jj]
                z = jnp.maximum(arow * dj + b1r, 0.0)
                g2v = g2v + jnp.where(lanes == jj, dj * jnp.sum(z * w2r), 0.0)
            g2b[pl.ds(jb * 16, 16)] = g2v
            return carry

        lax.fori_loop(0, zp // 16, nblk, 0)
        pltpu.sync_copy(g2b, g2sh.at[pl.ds(r0, zp)])

        @pl.when(cid == 0)
        def _():
            pltpu.sync_copy(g2b, g2_hbm.at[pl.ds(r0, zp)])

        plsc.subcore_barrier()

        def issue_i(c, b):
            pltpu.async_copy(edge_hbm.at[0, pl.ds(base + c * CH, CH)],
                             srci.at[b], sis.at[b])
            pltpu.async_copy(edge_hbm.at[1, pl.ds(base + c * CH, CH)],
                             dsti.at[b], sid_.at[b])

        def wait_i(c, b):
            pltpu.make_async_copy(edge_hbm.at[0, pl.ds(base + c * CH, CH)],
                                  srci.at[b], sis.at[b]).wait()
            pltpu.make_async_copy(edge_hbm.at[1, pl.ds(base + c * CH, CH)],
                                  dsti.at[b], sid_.at[b]).wait()

        def issue_g(c, b):
            pltpu.async_copy(g2sh.at[srci.at[b]], rows.at[b % 2],
                             sg.at[b % 2])

        def wait_g(c, b):
            pltpu.make_async_copy(g2sh.at[srci.at[b]], rows.at[b % 2],
                                  sg.at[b % 2]).wait()

        def issue_s(c, b):
            pltpu.async_copy(rows.at[b % 2], accsh.at[dsti.at[b]],
                             ss.at[b % 2], add=True)

        def wait_s(c, b):
            pltpu.make_async_copy(rows.at[b % 2], accsh.at[dsti.at[b]],
                                  ss.at[b % 2]).wait()

        def body(c, b, w_s2, w_g1, i_i2):
            if w_s2:
                wait_s(c - 2, (b - 2) % NBUF)
            wait_i(c, b)
            issue_g(c, b)
            if i_i2:
                issue_i(c + 2, (b + 2) % NBUF)
            if w_g1:
                wait_g(c - 1, (b - 1) % NBUF)
                issue_s(c - 1, (b - 1) % NBUF)

        def epilogue():
            wait_g(nch - 1, (nch - 1) % NBUF)
            issue_s(nch - 1, (nch - 1) % NBUF)
            wait_s(nch - 2, (nch - 2) % NBUF)
            wait_s(nch - 1, (nch - 1) % NBUF)

        _pipeline(nch, body, issue_i, epilogue)

        if tail:
            pltpu.sync_copy(edge_hbm.at[0, pl.ds(base + nch * CH, tail)], srct)
            pltpu.sync_copy(edge_hbm.at[1, pl.ds(base + nch * CH, tail)], dstt)
            pltpu.sync_copy(g2sh.at[srct], rowst)
            pltpu.sync_copy(rowst, accsh.at[dstt], add=True)

        plsc.subcore_barrier()
        pltpu.sync_copy(accsh.at[pl.ds(r0, zp)], stage)
        pltpu.sync_copy(stage, out_hbm.at[cid, pl.ds(r0, zp)])

    return k


def _mm_body(x_ref, w1_ref, hm_ref):
    hm_ref[...] = jnp.dot(x_ref[...], w1_ref[...],
                          preferred_element_type=jnp.float32)


def _tc3_body(aggp_ref, g2_ref, dis_ref, b2_ref, out_ref):
    s = aggp_ref[0] + aggp_ref[1] + g2_ref[...]
    out_ref[...] = jax.nn.sigmoid(dis_ref[...] * s + b2_ref[0, 0])[:, None]


def kernel(x, edge_index, W1, b1, W2, b2):
    n, d = x.shape
    h = W1.shape[1]
    e = edge_index.shape[1]

    blk = 1024
    npad = ((n + 1 + blk - 1) // blk) * blk
    grid = npad // blk

    xp = jnp.pad(x, ((0, npad - n), (0, 0)))

    degp = _deg_kernel(npad, e)(edge_index)

    hm = pl.pallas_call(
        _mm_body,
        grid=(grid,),
        in_specs=[
            pl.BlockSpec((blk, d), lambda i: (i, 0)),
            pl.BlockSpec((d, h), lambda i: (0, 0)),
        ],
        out_specs=pl.BlockSpec((blk, h), lambda i: (i, 0)),
        out_shape=jax.ShapeDtypeStruct((npad, h), jnp.float32),
    )(xp, W1)

    accp, dis = _agg1_kernel(npad, e, h)(
        hm.reshape(npad * h), degp, edge_index)

    agg2p, g2 = _layer2_kernel(npad, e, h)(
        accp.reshape(NC, npad * h), dis, b1, W2.reshape(h), edge_index)

    out = pl.pallas_call(
        _tc3_body,
        grid=(grid,),
        in_specs=[
            pl.BlockSpec((NC, blk), lambda i: (0, i)),
            pl.BlockSpec((blk,), lambda i: (i,)),
            pl.BlockSpec((blk,), lambda i: (i,)),
            pl.BlockSpec((1, 1), lambda i: (0, 0)),
        ],
        out_specs=pl.BlockSpec((blk, 1), lambda i: (i, 0)),
        out_shape=jax.ShapeDtypeStruct((n, 1), jnp.float32),
    )(agg2p, g2, dis, b2.reshape(1, 1))

    return out


# speedup vs baseline: 84.3629x; 1.0320x over previous
import functools

import jax
import jax.numpy as jnp
from jax import lax
from jax.experimental import pallas as pl
from jax.experimental.pallas import tpu as pltpu
from jax.experimental.pallas import tpu_sc as plsc

NC = 2
NS = 16
NW = NC * NS
CH = 128
NBUF = 4


def _sc_mesh():
    return plsc.VectorSubcoreMesh(
        core_axis_name="c", subcore_axis_name="s",
        num_cores=NC, num_subcores=NS)


def _sc_params(tc_tiling=False):
    return pltpu.CompilerParams(use_tc_tiling_on_sc=tc_tiling,
                                needs_layout_passes=False)


def _rsqrt16(x):
    yi = jnp.int32(0x5F3759DF) - (plsc.bitcast(x, jnp.int32) >> 1)
    y = plsc.bitcast(yi, jnp.float32)
    for _ in range(4):
        y = y * (1.5 - 0.5 * x * y * y)
    return y


def _fill(ref, size, value):
    v = jnp.full((16,), value, jnp.float32)

    def st(i, carry):
        ref[pl.ds(i * 16, 16)] = v
        return carry

    lax.fori_loop(0, size // 16, st, 0)


def _pipeline(nch, body, issue_i, epilogue):
    fg, rem = nch // NBUF, nch % NBUF
    issue_i(0, 0)
    issue_i(1, 1)
    body(0, 0, False, False, True)
    body(1, 1, False, True, True)
    body(2, 2, True, True, True)
    body(3, 3, True, True, True)
    steady_end = fg if rem else fg - 1

    def grp(g, carry):
        for b in range(NBUF):
            body(NBUF * g + b, b, True, True, True)
        return carry

    lax.fori_loop(1, steady_end, grp, 0)
    for c in range(NBUF * steady_end, nch):
        body(c, c % NBUF, True, True, c + 2 < nch)
    epilogue()


def _deg_kernel(npad, e):
    zp = npad // NS
    et = e // NW
    nch = et // CH
    tail = et - nch * CH

    @functools.partial(
        pl.kernel,
        out_type=jax.ShapeDtypeStruct((NC, npad), jnp.float32),
        mesh=_sc_mesh(),
        compiler_params=_sc_params(),
        scratch_types=[
            pltpu.VMEM((NBUF, CH), jnp.int32),
            pltpu.VMEM((16,), jnp.int32),
            pltpu.VMEM((CH,), jnp.float32),
            pltpu.VMEM((zp,), jnp.float32),
            pltpu.VMEM_SHARED((npad,), jnp.float32),
            pltpu.SemaphoreType.DMA((NBUF,)),
            pltpu.SemaphoreType.DMA((2,)),
        ],
    )
    def k(edge_hbm, out_hbm, dsti, dstt, onesb, stage, degsh, si, ss):
        cid = lax.axis_index("c")
        sid = lax.axis_index("s")
        wid = sid * NC + cid
        base = wid * et
        _fill(onesb, CH, 1.0)
        _fill(stage, zp, 0.0)
        pltpu.sync_copy(stage, degsh.at[pl.ds(sid * zp, zp)])
        plsc.subcore_barrier()

        def issue_i(c, b):
            pltpu.async_copy(edge_hbm.at[1, pl.ds(base + c * CH, CH)],
                             dsti.at[b], si.at[b])

        def wait_i(c, b):
            pltpu.make_async_copy(edge_hbm.at[1, pl.ds(base + c * CH, CH)],
                                  dsti.at[b], si.at[b]).wait()

        def issue_s(c, b):
            pltpu.async_copy(onesb, degsh.at[dsti.at[b]], ss.at[b % 2],
                             add=True)

        def wait_s(c, b):
            pltpu.make_async_copy(onesb, degsh.at[dsti.at[b]],
                                  ss.at[b % 2]).wait()

        def body(c, b, w_s2, w_g1, i_i2):
            if w_s2:
                wait_s(c - 2, (b - 2) % NBUF)
            wait_i(c, b)
            issue_s(c, b)
            if i_i2:
                issue_i(c + 2, (b + 2) % NBUF)

        def epilogue():
            wait_s(nch - 2, (nch - 2) % NBUF)
            wait_s(nch - 1, (nch - 1) % NBUF)

        _pipeline(nch, body, issue_i, epilogue)

        if tail:
            pltpu.sync_copy(edge_hbm.at[1, pl.ds(base + nch * CH, tail)], dstt)
            pltpu.sync_copy(onesb.at[pl.ds(0, tail)], degsh.at[dstt], add=True)

        plsc.subcore_barrier()
        pltpu.sync_copy(degsh.at[pl.ds(sid * zp, zp)], stage)
        pltpu.sync_copy(stage, out_hbm.at[cid, pl.ds(sid * zp, zp)])

    return k


def _agg1_kernel(npad, e, h):
    zp = npad // NS
    et = e // NW
    nch = et // CH
    tail = et - nch * CH

    @functools.partial(
        pl.kernel,
        out_type=[
            jax.ShapeDtypeStruct((NC, npad, h), jnp.float32),
            jax.ShapeDtypeStruct((npad,), jnp.float32),
        ],
        mesh=_sc_mesh(),
        compiler_params=_sc_params(),
        scratch_types=[
            pltpu.VMEM((NBUF, CH), jnp.int32),
            pltpu.VMEM((NBUF, CH), jnp.int32),
            pltpu.VMEM((16,), jnp.int32),
            pltpu.VMEM((16,), jnp.int32),
            pltpu.VMEM((2, CH, h), jnp.float32),
            pltpu.VMEM((16, h), jnp.float32),
            pltpu.VMEM((zp, h), jnp.float32),
            pltpu.VMEM((zp * h,), jnp.float32),
            pltpu.VMEM((16, h), jnp.float32),
            pltpu.VMEM((16, h), jnp.float32),
            pltpu.VMEM((zp,), jnp.float32),
            pltpu.VMEM((zp,), jnp.float32),
            pltpu.VMEM((zp,), jnp.float32),
            pltpu.VMEM_SHARED((npad, h), jnp.float32),
            pltpu.VMEM_SHARED((npad, h), jnp.float32),
            pltpu.SemaphoreType.DMA((NBUF,)),
            pltpu.SemaphoreType.DMA((NBUF,)),
            pltpu.SemaphoreType.DMA((2,)),
            pltpu.SemaphoreType.DMA((2,)),
        ],
    )
    def k(hf_hbm, degp_hbm, edge_hbm, out_hbm, dis_hbm,
          srci, dsti, srct, dstt, rows, rowst, stage, hbuf, tb, ztb,
          d0b, d1b, diss, g1sh, accsh, sis, sid_, sg, ss):
        cid = lax.axis_index("c")
        sid = lax.axis_index("s")
        wid = sid * NC + cid
        base = wid * et
        r0 = sid * zp

        pltpu.sync_copy(hf_hbm.at[pl.ds(r0 * h, zp * h)], hbuf)
        pltpu.sync_copy(degp_hbm.at[0, pl.ds(r0, zp)], d0b)
        pltpu.sync_copy(degp_hbm.at[1, pl.ds(r0, zp)], d1b)
        zrow = jnp.zeros((h,), jnp.float32)
        for jj in range(16):
            ztb[jj, :] = zrow

        def nblk(jb, carry):
            j16 = pl.ds(jb * 16, 16)
            dis16 = _rsqrt16(d0b[j16] + d1b[j16] + 1.0)
            diss[j16] = dis16
            for jj in range(16):
                o = (jb * 16 + jj) * h
                tb[jj, :] = hbuf[pl.ds(o, h)] * dis16[jj]
            rows16 = pl.ds(r0 + jb * 16, 16)
            pltpu.sync_copy(tb, g1sh.at[rows16, :])

            @pl.when(cid == 0)
            def _():
                pltpu.sync_copy(tb, accsh.at[rows16, :])

            @pl.when(cid != 0)
            def _():
                pltpu.sync_copy(ztb, accsh.at[rows16, :])

            return carry

        lax.fori_loop(0, zp // 16, nblk, 0)

        @pl.when(cid == 0)
        def _():
            pltpu.sync_copy(diss, dis_hbm.at[pl.ds(r0, zp)])

        plsc.subcore_barrier()

        def issue_i(c, b):
            pltpu.async_copy(edge_hbm.at[0, pl.ds(base + c * CH, CH)],
                             srci.at[b], sis.at[b])
            pltpu.async_copy(edge_hbm.at[1, pl.ds(base + c * CH, CH)],
                             dsti.at[b], sid_.at[b])

        def wait_i(c, b):
            pltpu.make_async_copy(edge_hbm.at[0, pl.ds(base + c * CH, CH)],
                                  srci.at[b], sis.at[b]).wait()
            pltpu.make_async_copy(edge_hbm.at[1, pl.ds(base + c * CH, CH)],
                                  dsti.at[b], sid_.at[b]).wait()

        def issue_g(c, b):
            pltpu.async_copy(g1sh.at[srci.at[b]], rows.at[b % 2],
                             sg.at[b % 2])

        def wait_g(c, b):
            pltpu.make_async_copy(g1sh.at[srci.at[b]], rows.at[b % 2],
                                  sg.at[b % 2]).wait()

        def issue_s(c, b):
            pltpu.async_copy(rows.at[b % 2], accsh.at[dsti.at[b]],
                             ss.at[b % 2], add=True)

        def wait_s(c, b):
            pltpu.make_async_copy(rows.at[b % 2], accsh.at[dsti.at[b]],
                                  ss.at[b % 2]).wait()

        def body(c, b, w_s2, w_g1, i_i2):
            if w_s2:
                wait_s(c - 2, (b - 2) % NBUF)
            wait_i(c, b)
            issue_g(c, b)
            if i_i2:
                issue_i(c + 2, (b + 2) % NBUF)
            if w_g1:
                wait_g(c - 1, (b - 1) % NBUF)
                issue_s(c - 1, (b - 1) % NBUF)

        def epilogue():
            wait_g(nch - 1, (nch - 1) % NBUF)
            issue_s(nch - 1, (nch - 1) % NBUF)
            wait_s(nch - 2, (nch - 2) % NBUF)
            wait_s(nch - 1, (nch - 1) % NBUF)

        _pipeline(nch, body, issue_i, epilogue)

        if tail:
            pltpu.sync_copy(edge_hbm.at[0, pl.ds(base + nch * CH, tail)], srct)
            pltpu.sync_copy(edge_hbm.at[1, pl.ds(base + nch * CH, tail)], dstt)
            pltpu.sync_copy(g1sh.at[srct], rowst)
            pltpu.sync_copy(rowst, accsh.at[dstt], add=True)

        plsc.subcore_barrier()
        pltpu.sync_copy(accsh.at[pl.ds(r0, zp), :], stage)
        pltpu.sync_copy(stage, out_hbm.at[cid, pl.ds(r0, zp)])

    return k


def _layer2_kernel(npad, e, h):
    zp = npad // NS
    et = e // NW
    nch = et // CH
    tail = et - nch * CH

    @functools.partial(
        pl.kernel,
        out_type=[
            jax.ShapeDtypeStruct((NC, npad), jnp.float32),
            jax.ShapeDtypeStruct((npad,), jnp.float32),
        ],
        mesh=_sc_mesh(),
        compiler_params=_sc_params(),
        scratch_types=[
            pltpu.VMEM((NBUF, CH), jnp.int32),
            pltpu.VMEM((NBUF, CH), jnp.int32),
            pltpu.VMEM((16,), jnp.int32),
            pltpu.VMEM((16,), jnp.int32),
            pltpu.VMEM((2, CH), jnp.float32),
            pltpu.VMEM((16,), jnp.float32),
            pltpu.VMEM((zp,), jnp.float32),
            pltpu.VMEM((zp * h,), jnp.float32),
            pltpu.VMEM((zp * h,), jnp.float32),
            pltpu.VMEM((zp,), jnp.float32),
            pltpu.VMEM((zp,), jnp.float32),
            pltpu.VMEM((h,), jnp.float32),
            pltpu.VMEM((h,), jnp.float32),
            pltpu.VMEM_SHARED((npad,), jnp.float32),
            pltpu.VMEM_SHARED((npad,), jnp.float32),
            pltpu.SemaphoreType.DMA((NBUF,)),
            pltpu.SemaphoreType.DMA((NBUF,)),
            pltpu.SemaphoreType.DMA((2,)),
            pltpu.SemaphoreType.DMA((2,)),
        ],
    )
    def k(accpf_hbm, dis_hbm, b1_hbm, w2_hbm, edge_hbm,
          out_hbm, g2_hbm,
          srci, dsti, srct, dstt, rows, rowst, stage,
          a0, a1, diss, g2b, b1v, w2v, g2sh, accsh,
          sis, sid_, sg, ss):
        cid = lax.axis_index("c")
        sid = lax.axis_index("s")
        wid = sid * NC + cid
        base = wid * et
        r0 = sid * zp

        pltpu.sync_copy(accpf_hbm.at[0, pl.ds(r0 * h, zp * h)], a0)
        pltpu.sync_copy(accpf_hbm.at[1, pl.ds(r0 * h, zp * h)], a1)
        pltpu.sync_copy(dis_hbm.at[pl.ds(r0, zp)], diss)
        pltpu.sync_copy(b1_hbm, b1v)
        pltpu.sync_copy(w2_hbm, w2v)
        _fill(stage, zp, 0.0)
        pltpu.sync_copy(stage, accsh.at[pl.ds(r0, zp)])
        b1r = b1v[...]
        w2r = w2v[...]
        lanes = lax.iota(jnp.int32, 16)

        def nblk(jb, carry):
            dis16 = diss[pl.ds(jb * 16, 16)]
            g2v = jnp.zeros((16,), jnp.float32)
            for jj in range(16):
                o = (jb * 16 + jj) * h
                arow = a0[pl.ds(o, h)] + a1[pl.ds(o, h)]
                dj = dis16[jj]
                z = jnp.maximum(arow * dj + b1r, 0.0)
                g2v = g2v + jnp.where(lanes == jj, dj * jnp.sum(z * w2r), 0.0)
            g2b[pl.ds(jb * 16, 16)] = g2v
            return carry

        lax.fori_loop(0, zp // 16, nblk, 0)
        pltpu.sync_copy(g2b, g2sh.at[pl.ds(r0, zp)])

        @pl.when(cid == 0)
        def _():
            pltpu.sync_copy(g2b, g2_hbm.at[pl.ds(r0, zp)])

        plsc.subcore_barrier()

        def issue_i(c, b):
            pltpu.async_copy(edge_hbm.at[0, pl.ds(base + c * CH, CH)],
                             srci.at[b], sis.at[b])
            pltpu.async_copy(edge_hbm.at[1, pl.ds(base + c * CH, CH)],
                             dsti.at[b], sid_.at[b])

        def wait_i(c, b):
            pltpu.make_async_copy(edge_hbm.at[0, pl.ds(base + c * CH, CH)],
                                  srci.at[b], sis.at[b]).wait()
            pltpu.make_async_copy(edge_hbm.at[1, pl.ds(base + c * CH, CH)],
                                  dsti.at[b], sid_.at[b]).wait()

        def issue_g(c, b):
            pltpu.async_copy(g2sh.at[srci.at[b]], rows.at[b % 2],
                             sg.at[b % 2])

        def wait_g(c, b):
            pltpu.make_async_copy(g2sh.at[srci.at[b]], rows.at[b % 2],
                                  sg.at[b % 2]).wait()

        def issue_s(c, b):
            pltpu.async_copy(rows.at[b % 2], accsh.at[dsti.at[b]],
                             ss.at[b % 2], add=True)

        def wait_s(c, b):
            pltpu.make_async_copy(rows.at[b % 2], accsh.at[dsti.at[b]],
                                  ss.at[b % 2]).wait()

        def body(c, b, w_s2, w_g1, i_i2):
            if w_s2:
                wait_s(c - 2, (b - 2) % NBUF)
            wait_i(c, b)
            issue_g(c, b)
            if i_i2:
                issue_i(c + 2, (b + 2) % NBUF)
            if w_g1:
                wait_g(c - 1, (b - 1) % NBUF)
                issue_s(c - 1, (b - 1) % NBUF)

        def epilogue():
            wait_g(nch - 1, (nch - 1) % NBUF)
            issue_s(nch - 1, (nch - 1) % NBUF)
            wait_s(nch - 2, (nch - 2) % NBUF)
            wait_s(nch - 1, (nch - 1) % NBUF)

        _pipeline(nch, body, issue_i, epilogue)

        if tail:
            pltpu.sync_copy(edge_hbm.at[0, pl.ds(base + nch * CH, tail)], srct)
            pltpu.sync_copy(edge_hbm.at[1, pl.ds(base + nch * CH, tail)], dstt)
            pltpu.sync_copy(g2sh.at[srct], rowst)
            pltpu.sync_copy(rowst, accsh.at[dstt], add=True)

        plsc.subcore_barrier()
        pltpu.sync_copy(accsh.at[pl.ds(r0, zp)], stage)
        pltpu.sync_copy(stage, out_hbm.at[cid, pl.ds(r0, zp)])

    return k


def _mm_body(x_ref, w1_ref, hm_ref):
    hm_ref[...] = jnp.dot(x_ref[...], w1_ref[...],
                          preferred_element_type=jnp.float32)


def _tc3_body(aggp_ref, g2_ref, dis_ref, b2_ref, out_ref):
    s = aggp_ref[0] + aggp_ref[1] + g2_ref[...]
    out_ref[...] = jax.nn.sigmoid(dis_ref[...] * s + b2_ref[0, 0])


def kernel(x, edge_index, W1, b1, W2, b2):
    n, d = x.shape
    h = W1.shape[1]
    e = edge_index.shape[1]

    blk = 1024
    npad = ((n + 1 + blk - 1) // blk) * blk
    grid = npad // blk

    xp = jnp.pad(x, ((0, npad - n), (0, 0)))

    degp = _deg_kernel(npad, e)(edge_index)

    hm = pl.pallas_call(
        _mm_body,
        grid=(grid,),
        in_specs=[
            pl.BlockSpec((blk, d), lambda i: (i, 0)),
            pl.BlockSpec((d, h), lambda i: (0, 0)),
        ],
        out_specs=pl.BlockSpec((blk, h), lambda i: (i, 0)),
        out_shape=jax.ShapeDtypeStruct((npad, h), jnp.float32),
    )(xp, W1)

    accp, dis = _agg1_kernel(npad, e, h)(
        hm.reshape(npad * h), degp, edge_index)

    agg2p, g2 = _layer2_kernel(npad, e, h)(
        accp.reshape(NC, npad * h), dis, b1, W2.reshape(h), edge_index)

    out = pl.pallas_call(
        _tc3_body,
        grid=(grid,),
        in_specs=[
            pl.BlockSpec((NC, blk), lambda i: (0, i)),
            pl.BlockSpec((blk,), lambda i: (i,)),
            pl.BlockSpec((blk,), lambda i: (i,)),
            pl.BlockSpec((1, 1), lambda i: (0, 0)),
        ],
        out_specs=pl.BlockSpec((blk,), lambda i: (i,)),
        out_shape=jax.ShapeDtypeStruct((n,), jnp.float32),
    )(agg2p, g2, dis, b2.reshape(1, 1))

    return out.reshape(n, 1)


# speedup vs baseline: 95.5369x; 1.1325x over previous
import functools

import jax
import jax.numpy as jnp
from jax import lax
from jax.experimental import pallas as pl
from jax.experimental.pallas import tpu as pltpu
from jax.experimental.pallas import tpu_sc as plsc

NC = 2
NS = 16
NW = NC * NS
CH = 128
NBUF = 4


def _sc_mesh():
    return plsc.VectorSubcoreMesh(
        core_axis_name="c", subcore_axis_name="s",
        num_cores=NC, num_subcores=NS)


def _sc_params(tc_tiling=False):
    return pltpu.CompilerParams(use_tc_tiling_on_sc=tc_tiling,
                                needs_layout_passes=False)


def _rsqrt16(x):
    yi = jnp.int32(0x5F3759DF) - (plsc.bitcast(x, jnp.int32) >> 1)
    y = plsc.bitcast(yi, jnp.float32)
    for _ in range(4):
        y = y * (1.5 - 0.5 * x * y * y)
    return y


def _fill(ref, size, value):
    v = jnp.full((16,), value, jnp.float32)

    def st(i, carry):
        ref[pl.ds(i * 16, 16)] = v
        return carry

    lax.fori_loop(0, size // 16, st, 0)


def _pipeline2(nch, body, issue_i):
    for c in range(4):
        issue_i(c, c % 8)
    for c in range(8):
        body(c, c % 8, c >= 4, c >= 2, True)
    fg8 = (nch - 8) // 8

    def grp(g, carry):
        for b in range(8):
            body(8 * g + b, b, True, True, True)
        return carry

    lax.fori_loop(1, 1 + fg8, grp, 0)
    for c in range(8 + 8 * fg8, nch):
        body(c, c % 8, True, True, c + 4 < nch)


def _pipeline(nch, body, issue_i, epilogue):
    fg, rem = nch // NBUF, nch % NBUF
    issue_i(0, 0)
    issue_i(1, 1)
    body(0, 0, False, False, True)
    body(1, 1, False, True, True)
    body(2, 2, True, True, True)
    body(3, 3, True, True, True)
    steady_end = fg if rem else fg - 1

    def grp(g, carry):
        for b in range(NBUF):
            body(NBUF * g + b, b, True, True, True)
        return carry

    lax.fori_loop(1, steady_end, grp, 0)
    for c in range(NBUF * steady_end, nch):
        body(c, c % NBUF, True, True, c + 2 < nch)
    epilogue()


def _deg_kernel(npad, e):
    zp = npad // NS
    et = e // NW
    nch = et // CH
    tail = et - nch * CH

    @functools.partial(
        pl.kernel,
        out_type=jax.ShapeDtypeStruct((NC, npad), jnp.float32),
        mesh=_sc_mesh(),
        compiler_params=_sc_params(),
        scratch_types=[
            pltpu.VMEM((NBUF, CH), jnp.int32),
            pltpu.VMEM((16,), jnp.int32),
            pltpu.VMEM((CH,), jnp.float32),
            pltpu.VMEM((zp,), jnp.float32),
            pltpu.VMEM_SHARED((npad,), jnp.float32),
            pltpu.SemaphoreType.DMA((NBUF,)),
            pltpu.SemaphoreType.DMA((2,)),
        ],
    )
    def k(edge_hbm, out_hbm, dsti, dstt, onesb, stage, degsh, si, ss):
        cid = lax.axis_index("c")
        sid = lax.axis_index("s")
        wid = sid * NC + cid
        base = wid * et
        _fill(onesb, CH, 1.0)
        _fill(stage, zp, 0.0)
        pltpu.sync_copy(stage, degsh.at[pl.ds(sid * zp, zp)])
        plsc.subcore_barrier()

        def issue_i(c, b):
            pltpu.async_copy(edge_hbm.at[1, pl.ds(base + c * CH, CH)],
                             dsti.at[b], si.at[b])

        def wait_i(c, b):
            pltpu.make_async_copy(edge_hbm.at[1, pl.ds(base + c * CH, CH)],
                                  dsti.at[b], si.at[b]).wait()

        def issue_s(c, b):
            pltpu.async_copy(onesb, degsh.at[dsti.at[b]], ss.at[b % 2],
                             add=True)

        def wait_s(c, b):
            pltpu.make_async_copy(onesb, degsh.at[dsti.at[b]],
                                  ss.at[b % 2]).wait()

        def body(c, b, w_s2, w_g1, i_i2):
            if w_s2:
                wait_s(c - 2, (b - 2) % NBUF)
            wait_i(c, b)
            issue_s(c, b)
            if i_i2:
                issue_i(c + 2, (b + 2) % NBUF)

        def epilogue():
            wait_s(nch - 2, (nch - 2) % NBUF)
            wait_s(nch - 1, (nch - 1) % NBUF)

        _pipeline(nch, body, issue_i, epilogue)

        if tail:
            pltpu.sync_copy(edge_hbm.at[1, pl.ds(base + nch * CH, tail)], dstt)
            pltpu.sync_copy(onesb.at[pl.ds(0, tail)], degsh.at[dstt], add=True)

        plsc.subcore_barrier()
        pltpu.sync_copy(degsh.at[pl.ds(sid * zp, zp)], stage)
        pltpu.sync_copy(stage, out_hbm.at[cid, pl.ds(sid * zp, zp)])

    return k


def _agg1_kernel(npad, e, h):
    zp = npad // NS
    et = e // NW
    nch = et // CH
    tail = et - nch * CH

    @functools.partial(
        pl.kernel,
        out_type=[
            jax.ShapeDtypeStruct((NC, npad, h), jnp.float32),
            jax.ShapeDtypeStruct((npad,), jnp.float32),
        ],
        mesh=_sc_mesh(),
        compiler_params=_sc_params(),
        scratch_types=[
            pltpu.VMEM((8, CH), jnp.int32),
            pltpu.VMEM((8, CH), jnp.int32),
            pltpu.VMEM((16,), jnp.int32),
            pltpu.VMEM((16,), jnp.int32),
            pltpu.VMEM((4, CH, h), jnp.float32),
            pltpu.VMEM((16, h), jnp.float32),
            pltpu.VMEM((zp, h), jnp.float32),
            pltpu.VMEM((zp * h,), jnp.float32),
            pltpu.VMEM((16, h), jnp.float32),
            pltpu.VMEM((16, h), jnp.float32),
            pltpu.VMEM((zp,), jnp.float32),
            pltpu.VMEM((zp,), jnp.float32),
            pltpu.VMEM((zp,), jnp.float32),
            pltpu.VMEM_SHARED((npad, h), jnp.float32),
            pltpu.VMEM_SHARED((npad, h), jnp.float32),
            pltpu.SemaphoreType.DMA((8,)),
            pltpu.SemaphoreType.DMA((8,)),
            pltpu.SemaphoreType.DMA((4,)),
            pltpu.SemaphoreType.DMA((4,)),
        ],
    )
    def k(hf_hbm, degp_hbm, edge_hbm, out_hbm, dis_hbm,
          srci, dsti, srct, dstt, rows, rowst, stage, hbuf, tb, ztb,
          d0b, d1b, diss, g1sh, accsh, sis, sid_, sg, ss):
        cid = lax.axis_index("c")
        sid = lax.axis_index("s")
        wid = sid * NC + cid
        base = wid * et
        r0 = sid * zp

        pltpu.sync_copy(hf_hbm.at[pl.ds(r0 * h, zp * h)], hbuf)
        pltpu.sync_copy(degp_hbm.at[0, pl.ds(r0, zp)], d0b)
        pltpu.sync_copy(degp_hbm.at[1, pl.ds(r0, zp)], d1b)
        zrow = jnp.zeros((h,), jnp.float32)
        for jj in range(16):
            ztb[jj, :] = zrow

        def nblk(jb, carry):
            j16 = pl.ds(jb * 16, 16)
            dis16 = _rsqrt16(d0b[j16] + d1b[j16] + 1.0)
            diss[j16] = dis16
            for jj in range(16):
                o = (jb * 16 + jj) * h
                tb[jj, :] = hbuf[pl.ds(o, h)] * dis16[jj]
            rows16 = pl.ds(r0 + jb * 16, 16)
            pltpu.sync_copy(tb, g1sh.at[rows16, :])

            @pl.when(cid == 0)
            def _():
                pltpu.sync_copy(tb, accsh.at[rows16, :])

            @pl.when(cid != 0)
            def _():
                pltpu.sync_copy(ztb, accsh.at[rows16, :])

            return carry

        lax.fori_loop(0, zp // 16, nblk, 0)

        @pl.when(cid == 0)
        def _():
            pltpu.sync_copy(diss, dis_hbm.at[pl.ds(r0, zp)])

        plsc.subcore_barrier()

        def issue_i(c, b):
            pltpu.async_copy(edge_hbm.at[0, pl.ds(base + c * CH, CH)],
                             srci.at[b], sis.at[b])
            pltpu.async_copy(edge_hbm.at[1, pl.ds(base + c * CH, CH)],
                             dsti.at[b], sid_.at[b])

        def wait_i(c, b):
            pltpu.make_async_copy(edge_hbm.at[0, pl.ds(base + c * CH, CH)],
                                  srci.at[b], sis.at[b]).wait()
            pltpu.make_async_copy(edge_hbm.at[1, pl.ds(base + c * CH, CH)],
                                  dsti.at[b], sid_.at[b]).wait()

        def issue_g(c, b):
            pltpu.async_copy(g1sh.at[srci.at[b]], rows.at[b % 4],
                             sg.at[b % 4])

        def wait_g(c, b):
            pltpu.make_async_copy(g1sh.at[srci.at[b]], rows.at[b % 4],
                                  sg.at[b % 4]).wait()

        def issue_s(c, b):
            pltpu.async_copy(rows.at[b % 4], accsh.at[dsti.at[b]],
                             ss.at[b % 4], add=True)

        def wait_s(c, b):
            pltpu.make_async_copy(rows.at[b % 4], accsh.at[dsti.at[b]],
                                  ss.at[b % 4]).wait()

        def body(c, b8, w_s4, w_g2, i_i4):
            if w_s4:
                wait_s(c - 4, (b8 - 4) % 8)
            wait_i(c, b8)
            issue_g(c, b8)
            if i_i4:
                issue_i(c + 4, (b8 + 4) % 8)
            if w_g2:
                wait_g(c - 2, (b8 - 2) % 8)
                issue_s(c - 2, (b8 - 2) % 8)

        _pipeline2(nch, body, issue_i)
        for cc in (nch - 2, nch - 1):
            wait_g(cc, cc % 8)
            issue_s(cc, cc % 8)
        for cc in range(nch - 4, nch):
            wait_s(cc, cc % 8)

        if tail:
            pltpu.sync_copy(edge_hbm.at[0, pl.ds(base + nch * CH, tail)], srct)
            pltpu.sync_copy(edge_hbm.at[1, pl.ds(base + nch * CH, tail)], dstt)
            pltpu.sync_copy(g1sh.at[srct], rowst)
            pltpu.sync_copy(rowst, accsh.at[dstt], add=True)

        plsc.subcore_barrier()
        pltpu.sync_copy(accsh.at[pl.ds(r0, zp), :], stage)
        pltpu.sync_copy(stage, out_hbm.at[cid, pl.ds(r0, zp)])

    return k


def _layer2_kernel(npad, e, h):
    zp = npad // NS
    et = e // NW
    nch = et // CH
    tail = et - nch * CH

    @functools.partial(
        pl.kernel,
        out_type=[
            jax.ShapeDtypeStruct((NC, npad), jnp.float32),
            jax.ShapeDtypeStruct((npad,), jnp.float32),
        ],
        mesh=_sc_mesh(),
        compiler_params=_sc_params(),
        scratch_types=[
            pltpu.VMEM((8, CH), jnp.int32),
            pltpu.VMEM((8, CH), jnp.int32),
            pltpu.VMEM((16,), jnp.int32),
            pltpu.VMEM((16,), jnp.int32),
            pltpu.VMEM((4, CH), jnp.float32),
            pltpu.VMEM((16,), jnp.float32),
            pltpu.VMEM((zp,), jnp.float32),
            pltpu.VMEM((zp * h,), jnp.float32),
            pltpu.VMEM((zp * h,), jnp.float32),
            pltpu.VMEM((zp,), jnp.float32),
            pltpu.VMEM((zp,), jnp.float32),
            pltpu.VMEM((h,), jnp.float32),
            pltpu.VMEM((h,), jnp.float32),
            pltpu.VMEM_SHARED((npad,), jnp.float32),
            pltpu.VMEM_SHARED((npad,), jnp.float32),
            pltpu.SemaphoreType.DMA((8,)),
            pltpu.SemaphoreType.DMA((8,)),
            pltpu.SemaphoreType.DMA((4,)),
            pltpu.SemaphoreType.DMA((4,)),
        ],
    )
    def k(accpf_hbm, dis_hbm, b1_hbm, w2_hbm, edge_hbm,
          out_hbm, g2_hbm,
          srci, dsti, srct, dstt, rows, rowst, stage,
          a0, a1, diss, g2b, b1v, w2v, g2sh, accsh,
          sis, sid_, sg, ss):
        cid = lax.axis_index("c")
        sid = lax.axis_index("s")
        wid = sid * NC + cid
        base = wid * et
        r0 = sid * zp

        pltpu.sync_copy(accpf_hbm.at[0, pl.ds(r0 * h, zp * h)], a0)
        pltpu.sync_copy(accpf_hbm.at[1, pl.ds(r0 * h, zp * h)], a1)
        pltpu.sync_copy(dis_hbm.at[pl.ds(r0, zp)], diss)
        pltpu.sync_copy(b1_hbm, b1v)
        pltpu.sync_copy(w2_hbm, w2v)
        _fill(stage, zp, 0.0)
        pltpu.sync_copy(stage, accsh.at[pl.ds(r0, zp)])
        b1r = b1v[...]
        w2r = w2v[...]
        lanes = lax.iota(jnp.int32, 16)

        def nblk(jb, carry):
            dis16 = diss[pl.ds(jb * 16, 16)]
            g2v = jnp.zeros((16,), jnp.float32)
            for jj in range(16):
                o = (jb * 16 + jj) * h
                arow = a0[pl.ds(o, h)] + a1[pl.ds(o, h)]
                dj = dis16[jj]
                z = jnp.maximum(arow * dj + b1r, 0.0)
                g2v = g2v + jnp.where(lanes == jj, dj * jnp.sum(z * w2r), 0.0)
            g2b[pl.ds(jb * 16, 16)] = g2v
            return carry

        lax.fori_loop(0, zp // 16, nblk, 0)
        pltpu.sync_copy(g2b, g2sh.at[pl.ds(r0, zp)])

        @pl.when(cid == 0)
        def _():
            pltpu.sync_copy(g2b, g2_hbm.at[pl.ds(r0, zp)])

        plsc.subcore_barrier()

        def issue_i(c, b):
            pltpu.async_copy(edge_hbm.at[0, pl.ds(base + c * CH, CH)],
                             srci.at[b], sis.at[b])
            pltpu.async_copy(edge_hbm.at[1, pl.ds(base + c * CH, CH)],
                             dsti.at[b], sid_.at[b])

        def wait_i(c, b):
            pltpu.make_async_copy(edge_hbm.at[0, pl.ds(base + c * CH, CH)],
                                  srci.at[b], sis.at[b]).wait()
            pltpu.make_async_copy(edge_hbm.at[1, pl.ds(base + c * CH, CH)],
                                  dsti.at[b], sid_.at[b]).wait()

        def issue_g(c, b):
            pltpu.async_copy(g2sh.at[srci.at[b]], rows.at[b % 4],
                             sg.at[b % 4])

        def wait_g(c, b):
            pltpu.make_async_copy(g2sh.at[srci.at[b]], rows.at[b % 4],
                                  sg.at[b % 4]).wait()

        def issue_s(c, b):
            pltpu.async_copy(rows.at[b % 4], accsh.at[dsti.at[b]],
                             ss.at[b % 4], add=True)

        def wait_s(c, b):
            pltpu.make_async_copy(rows.at[b % 4], accsh.at[dsti.at[b]],
                                  ss.at[b % 4]).wait()

        def body(c, b8, w_s4, w_g2, i_i4):
            if w_s4:
                wait_s(c - 4, (b8 - 4) % 8)
            wait_i(c, b8)
            issue_g(c, b8)
            if i_i4:
                issue_i(c + 4, (b8 + 4) % 8)
            if w_g2:
                wait_g(c - 2, (b8 - 2) % 8)
                issue_s(c - 2, (b8 - 2) % 8)

        _pipeline2(nch, body, issue_i)
        for cc in (nch - 2, nch - 1):
            wait_g(cc, cc % 8)
            issue_s(cc, cc % 8)
        for cc in range(nch - 4, nch):
            wait_s(cc, cc % 8)

        if tail:
            pltpu.sync_copy(edge_hbm.at[0, pl.ds(base + nch * CH, tail)], srct)
            pltpu.sync_copy(edge_hbm.at[1, pl.ds(base + nch * CH, tail)], dstt)
            pltpu.sync_copy(g2sh.at[srct], rowst)
            pltpu.sync_copy(rowst, accsh.at[dstt], add=True)

        plsc.subcore_barrier()
        pltpu.sync_copy(accsh.at[pl.ds(r0, zp)], stage)
        pltpu.sync_copy(stage, out_hbm.at[cid, pl.ds(r0, zp)])

    return k


def _mm_body(x_ref, w1_ref, hm_ref):
    hm_ref[...] = jnp.dot(x_ref[...], w1_ref[...],
                          preferred_element_type=jnp.float32)


def _tc3_body(aggp_ref, g2_ref, dis_ref, b2_ref, out_ref):
    s = aggp_ref[0] + aggp_ref[1] + g2_ref[...]
    out_ref[...] = jax.nn.sigmoid(dis_ref[...] * s + b2_ref[0, 0])


def kernel(x, edge_index, W1, b1, W2, b2):
    n, d = x.shape
    h = W1.shape[1]
    e = edge_index.shape[1]

    blk = 1024
    npad = ((n + 1 + blk - 1) // blk) * blk
    grid = npad // blk

    xp = jnp.pad(x, ((0, npad - n), (0, 0)))

    degp = _deg_kernel(npad, e)(edge_index)

    hm = pl.pallas_call(
        _mm_body,
        grid=(grid,),
        in_specs=[
            pl.BlockSpec((blk, d), lambda i: (i, 0)),
            pl.BlockSpec((d, h), lambda i: (0, 0)),
        ],
        out_specs=pl.BlockSpec((blk, h), lambda i: (i, 0)),
        out_shape=jax.ShapeDtypeStruct((npad, h), jnp.float32),
    )(xp, W1)

    accp, dis = _agg1_kernel(npad, e, h)(
        hm.reshape(npad * h), degp, edge_index)

    agg2p, g2 = _layer2_kernel(npad, e, h)(
        accp.reshape(NC, npad * h), dis, b1, W2.reshape(h), edge_index)

    out = pl.pallas_call(
        _tc3_body,
        grid=(grid,),
        in_specs=[
            pl.BlockSpec((NC, blk), lambda i: (0, i)),
            pl.BlockSpec((blk,), lambda i: (i,)),
            pl.BlockSpec((blk,), lambda i: (i,)),
            pl.BlockSpec((1, 1), lambda i: (0, 0)),
        ],
        out_specs=pl.BlockSpec((blk,), lambda i: (i,)),
        out_shape=jax.ShapeDtypeStruct((n,), jnp.float32),
    )(agg2p, g2, dis, b2.reshape(1, 1))

    return out.reshape(n, 1)


# speedup vs baseline: 103.4052x; 1.0824x over previous
import functools

import jax
import jax.numpy as jnp
from jax import lax
from jax.experimental import pallas as pl
from jax.experimental.pallas import tpu as pltpu
from jax.experimental.pallas import tpu_sc as plsc

NC = 2
NS = 16
NW = NC * NS
CH = 128
NBUF = 4


def _sc_mesh():
    return plsc.VectorSubcoreMesh(
        core_axis_name="c", subcore_axis_name="s",
        num_cores=NC, num_subcores=NS)


def _sc_params(tc_tiling=False):
    return pltpu.CompilerParams(use_tc_tiling_on_sc=tc_tiling,
                                needs_layout_passes=False)


def _rsqrt16(x):
    yi = jnp.int32(0x5F3759DF) - (plsc.bitcast(x, jnp.int32) >> 1)
    y = plsc.bitcast(yi, jnp.float32)
    for _ in range(4):
        y = y * (1.5 - 0.5 * x * y * y)
    return y


def _fill(ref, size, value):
    v = jnp.full((16,), value, jnp.float32)

    def st(i, carry):
        ref[pl.ds(i * 16, 16)] = v
        return carry

    lax.fori_loop(0, size // 16, st, 0)


def _pipeline2(nch, body, issue_i):
    for c in range(4):
        issue_i(c, c % 8)
    for c in range(8):
        body(c, c % 8, c >= 4, c >= 2, True)
    fg8 = (nch - 8) // 8

    def grp(g, carry):
        for b in range(8):
            body(8 * g + b, b, True, True, True)
        return carry

    lax.fori_loop(1, 1 + fg8, grp, 0)
    for c in range(8 + 8 * fg8, nch):
        body(c, c % 8, True, True, c + 4 < nch)


def _pipeline(nch, body, issue_i, epilogue):
    fg, rem = nch // NBUF, nch % NBUF
    issue_i(0, 0)
    issue_i(1, 1)
    body(0, 0, False, False, True)
    body(1, 1, False, True, True)
    body(2, 2, True, True, True)
    body(3, 3, True, True, True)
    steady_end = fg if rem else fg - 1

    def grp(g, carry):
        for b in range(NBUF):
            body(NBUF * g + b, b, True, True, True)
        return carry

    lax.fori_loop(1, steady_end, grp, 0)
    for c in range(NBUF * steady_end, nch):
        body(c, c % NBUF, True, True, c + 2 < nch)
    epilogue()


def _deg_kernel(npad, e):
    zp = npad // NS
    et = e // NW
    nch = et // CH
    tail = et - nch * CH

    @functools.partial(
        pl.kernel,
        out_type=jax.ShapeDtypeStruct((NC, npad), jnp.float32),
        mesh=_sc_mesh(),
        compiler_params=_sc_params(),
        scratch_types=[
            pltpu.VMEM((8, CH), jnp.int32),
            pltpu.VMEM((16,), jnp.int32),
            pltpu.VMEM((CH,), jnp.float32),
            pltpu.VMEM((zp,), jnp.float32),
            pltpu.VMEM_SHARED((npad,), jnp.float32),
            pltpu.SemaphoreType.DMA((8,)),
            pltpu.SemaphoreType.DMA((4,)),
        ],
    )
    def k(edge_hbm, out_hbm, dsti, dstt, onesb, stage, degsh, si, ss):
        cid = lax.axis_index("c")
        sid = lax.axis_index("s")
        wid = sid * NC + cid
        base = wid * et
        _fill(onesb, CH, 1.0)
        _fill(stage, zp, 0.0)
        pltpu.sync_copy(stage, degsh.at[pl.ds(sid * zp, zp)])
        plsc.subcore_barrier()

        def issue_i(c, b):
            pltpu.async_copy(edge_hbm.at[1, pl.ds(base + c * CH, CH)],
                             dsti.at[b], si.at[b])

        def wait_i(c, b):
            pltpu.make_async_copy(edge_hbm.at[1, pl.ds(base + c * CH, CH)],
                                  dsti.at[b], si.at[b]).wait()

        def issue_s(c, b):
            pltpu.async_copy(onesb, degsh.at[dsti.at[b]], ss.at[b % 4],
                             add=True)

        def wait_s(c, b):
            pltpu.make_async_copy(onesb, degsh.at[dsti.at[b]],
                                  ss.at[b % 4]).wait()

        def body(c, b8, w_s4, w_g2, i_i4):
            if w_s4:
                wait_s(c - 4, (b8 - 4) % 8)
            wait_i(c, b8)
            issue_s(c, b8)
            if i_i4:
                issue_i(c + 4, (b8 + 4) % 8)

        _pipeline2(nch, body, issue_i)
        for cc in range(nch - 4, nch):
            wait_s(cc, cc % 8)

        if tail:
            pltpu.sync_copy(edge_hbm.at[1, pl.ds(base + nch * CH, tail)], dstt)
            pltpu.sync_copy(onesb.at[pl.ds(0, tail)], degsh.at[dstt], add=True)

        plsc.subcore_barrier()
        pltpu.sync_copy(degsh.at[pl.ds(sid * zp, zp)], stage)
        pltpu.sync_copy(stage, out_hbm.at[cid, pl.ds(sid * zp, zp)])

    return k


def _agg1_kernel(npad, e, h):
    zp = npad // NS
    et = e // NW
    nch = et // CH
    tail = et - nch * CH

    @functools.partial(
        pl.kernel,
        out_type=[
            jax.ShapeDtypeStruct((NC, npad, h), jnp.float32),
            jax.ShapeDtypeStruct((npad,), jnp.float32),
        ],
        mesh=_sc_mesh(),
        compiler_params=_sc_params(),
        scratch_types=[
            pltpu.VMEM((8, CH), jnp.int32),
            pltpu.VMEM((8, CH), jnp.int32),
            pltpu.VMEM((16,), jnp.int32),
            pltpu.VMEM((16,), jnp.int32),
            pltpu.VMEM((4, CH, h), jnp.float32),
            pltpu.VMEM((16, h), jnp.float32),
            pltpu.VMEM((zp, h), jnp.float32),
            pltpu.VMEM((zp * h,), jnp.float32),
            pltpu.VMEM((16, h), jnp.float32),
            pltpu.VMEM((16, h), jnp.float32),
            pltpu.VMEM((zp,), jnp.float32),
            pltpu.VMEM((zp,), jnp.float32),
            pltpu.VMEM((zp,), jnp.float32),
            pltpu.VMEM_SHARED((npad, h), jnp.float32),
            pltpu.VMEM_SHARED((npad, h), jnp.float32),
            pltpu.SemaphoreType.DMA((8,)),
            pltpu.SemaphoreType.DMA((8,)),
            pltpu.SemaphoreType.DMA((4,)),
            pltpu.SemaphoreType.DMA((4,)),
        ],
    )
    def k(hf_hbm, degp_hbm, edge_hbm, out_hbm, dis_hbm,
          srci, dsti, srct, dstt, rows, rowst, stage, hbuf, tb, ztb,
          d0b, d1b, diss, g1sh, accsh, sis, sid_, sg, ss):
        cid = lax.axis_index("c")
        sid = lax.axis_index("s")
        wid = sid * NC + cid
        base = wid * et
        r0 = sid * zp

        pltpu.sync_copy(hf_hbm.at[pl.ds(r0 * h, zp * h)], hbuf)
        pltpu.sync_copy(degp_hbm.at[0, pl.ds(r0, zp)], d0b)
        pltpu.sync_copy(degp_hbm.at[1, pl.ds(r0, zp)], d1b)
        zrow = jnp.zeros((h,), jnp.float32)
        for jj in range(16):
            ztb[jj, :] = zrow

        def nblk(jb, carry):
            j16 = pl.ds(jb * 16, 16)
            dis16 = _rsqrt16(d0b[j16] + d1b[j16] + 1.0)
            diss[j16] = dis16
            for jj in range(16):
                o = (jb * 16 + jj) * h
                tb[jj, :] = hbuf[pl.ds(o, h)] * dis16[jj]
            rows16 = pl.ds(r0 + jb * 16, 16)
            pltpu.sync_copy(tb, g1sh.at[rows16, :])

            @pl.when(cid == 0)
            def _():
                pltpu.sync_copy(tb, accsh.at[rows16, :])

            @pl.when(cid != 0)
            def _():
                pltpu.sync_copy(ztb, accsh.at[rows16, :])

            return carry

        lax.fori_loop(0, zp // 16, nblk, 0)

        @pl.when(cid == 0)
        def _():
            pltpu.sync_copy(diss, dis_hbm.at[pl.ds(r0, zp)])

        plsc.subcore_barrier()

        def issue_i(c, b):
            pltpu.async_copy(edge_hbm.at[0, pl.ds(base + c * CH, CH)],
                             srci.at[b], sis.at[b])
            pltpu.async_copy(edge_hbm.at[1, pl.ds(base + c * CH, CH)],
                             dsti.at[b], sid_.at[b])

        def wait_i(c, b):
            pltpu.make_async_copy(edge_hbm.at[0, pl.ds(base + c * CH, CH)],
                                  srci.at[b], sis.at[b]).wait()
            pltpu.make_async_copy(edge_hbm.at[1, pl.ds(base + c * CH, CH)],
                                  dsti.at[b], sid_.at[b]).wait()

        def issue_g(c, b):
            pltpu.async_copy(g1sh.at[srci.at[b]], rows.at[b % 4],
                             sg.at[b % 4])

        def wait_g(c, b):
            pltpu.make_async_copy(g1sh.at[srci.at[b]], rows.at[b % 4],
                                  sg.at[b % 4]).wait()

        def issue_s(c, b):
            pltpu.async_copy(rows.at[b % 4], accsh.at[dsti.at[b]],
                             ss.at[b % 4], add=True)

        def wait_s(c, b):
            pltpu.make_async_copy(rows.at[b % 4], accsh.at[dsti.at[b]],
                                  ss.at[b % 4]).wait()

        def body(c, b8, w_s4, w_g2, i_i4):
            if w_s4:
                wait_s(c - 4, (b8 - 4) % 8)
            wait_i(c, b8)
            issue_g(c, b8)
            if i_i4:
                issue_i(c + 4, (b8 + 4) % 8)
            if w_g2:
                wait_g(c - 2, (b8 - 2) % 8)
                issue_s(c - 2, (b8 - 2) % 8)

        _pipeline2(nch, body, issue_i)
        for cc in (nch - 2, nch - 1):
            wait_g(cc, cc % 8)
            issue_s(cc, cc % 8)
        for cc in range(nch - 4, nch):
            wait_s(cc, cc % 8)

        if tail:
            pltpu.sync_copy(edge_hbm.at[0, pl.ds(base + nch * CH, tail)], srct)
            pltpu.sync_copy(edge_hbm.at[1, pl.ds(base + nch * CH, tail)], dstt)
            pltpu.sync_copy(g1sh.at[srct], rowst)
            pltpu.sync_copy(rowst, accsh.at[dstt], add=True)

        plsc.subcore_barrier()
        pltpu.sync_copy(accsh.at[pl.ds(r0, zp), :], stage)
        pltpu.sync_copy(stage, out_hbm.at[cid, pl.ds(r0, zp)])

    return k


def _layer2_kernel(npad, e, h):
    zp = npad // NS
    et = e // NW
    nch = et // CH
    tail = et - nch * CH

    @functools.partial(
        pl.kernel,
        out_type=[
            jax.ShapeDtypeStruct((NC, npad), jnp.float32),
            jax.ShapeDtypeStruct((npad,), jnp.float32),
        ],
        mesh=_sc_mesh(),
        compiler_params=_sc_params(),
        scratch_types=[
            pltpu.VMEM((8, CH), jnp.int32),
            pltpu.VMEM((8, CH), jnp.int32),
            pltpu.VMEM((16,), jnp.int32),
            pltpu.VMEM((16,), jnp.int32),
            pltpu.VMEM((4, CH), jnp.float32),
            pltpu.VMEM((16,), jnp.float32),
            pltpu.VMEM((zp,), jnp.float32),
            pltpu.VMEM((zp * h,), jnp.float32),
            pltpu.VMEM((zp * h,), jnp.float32),
            pltpu.VMEM((zp,), jnp.float32),
            pltpu.VMEM((zp,), jnp.float32),
            pltpu.VMEM((h,), jnp.float32),
            pltpu.VMEM((h,), jnp.float32),
            pltpu.VMEM_SHARED((npad,), jnp.float32),
            pltpu.VMEM_SHARED((npad,), jnp.float32),
            pltpu.SemaphoreType.DMA((8,)),
            pltpu.SemaphoreType.DMA((8,)),
            pltpu.SemaphoreType.DMA((4,)),
            pltpu.SemaphoreType.DMA((4,)),
        ],
    )
    def k(accpf_hbm, dis_hbm, b1_hbm, w2_hbm, edge_hbm,
          out_hbm, g2_hbm,
          srci, dsti, srct, dstt, rows, rowst, stage,
          a0, a1, diss, g2b, b1v, w2v, g2sh, accsh,
          sis, sid_, sg, ss):
        cid = lax.axis_index("c")
        sid = lax.axis_index("s")
        wid = sid * NC + cid
        base = wid * et
        r0 = sid * zp

        pltpu.sync_copy(accpf_hbm.at[0, pl.ds(r0 * h, zp * h)], a0)
        pltpu.sync_copy(accpf_hbm.at[1, pl.ds(r0 * h, zp * h)], a1)
        pltpu.sync_copy(dis_hbm.at[pl.ds(r0, zp)], diss)
        pltpu.sync_copy(b1_hbm, b1v)
        pltpu.sync_copy(w2_hbm, w2v)
        _fill(stage, zp, 0.0)
        pltpu.sync_copy(stage, accsh.at[pl.ds(r0, zp)])
        b1r = b1v[...]
        w2r = w2v[...]
        lanes = lax.iota(jnp.int32, 16)

        def nblk(jb, carry):
            dis16 = diss[pl.ds(jb * 16, 16)]
            g2v = jnp.zeros((16,), jnp.float32)
            for jj in range(16):
                o = (jb * 16 + jj) * h
                arow = a0[pl.ds(o, h)] + a1[pl.ds(o, h)]
                dj = dis16[jj]
                z = jnp.maximum(arow * dj + b1r, 0.0)
                g2v = g2v + jnp.where(lanes == jj, dj * jnp.sum(z * w2r), 0.0)
            g2b[pl.ds(jb * 16, 16)] = g2v
            return carry

        lax.fori_loop(0, zp // 16, nblk, 0)
        pltpu.sync_copy(g2b, g2sh.at[pl.ds(r0, zp)])

        @pl.when(cid == 0)
        def _():
            pltpu.sync_copy(g2b, g2_hbm.at[pl.ds(r0, zp)])

        plsc.subcore_barrier()

        def issue_i(c, b):
            pltpu.async_copy(edge_hbm.at[0, pl.ds(base + c * CH, CH)],
                             srci.at[b], sis.at[b])
            pltpu.async_copy(edge_hbm.at[1, pl.ds(base + c * CH, CH)],
                             dsti.at[b], sid_.at[b])

        def wait_i(c, b):
            pltpu.make_async_copy(edge_hbm.at[0, pl.ds(base + c * CH, CH)],
                                  srci.at[b], sis.at[b]).wait()
            pltpu.make_async_copy(edge_hbm.at[1, pl.ds(base + c * CH, CH)],
                                  dsti.at[b], sid_.at[b]).wait()

        def issue_g(c, b):
            pltpu.async_copy(g2sh.at[srci.at[b]], rows.at[b % 4],
                             sg.at[b % 4])

        def wait_g(c, b):
            pltpu.make_async_copy(g2sh.at[srci.at[b]], rows.at[b % 4],
                                  sg.at[b % 4]).wait()

        def issue_s(c, b):
            pltpu.async_copy(rows.at[b % 4], accsh.at[dsti.at[b]],
                             ss.at[b % 4], add=True)

        def wait_s(c, b):
            pltpu.make_async_copy(rows.at[b % 4], accsh.at[dsti.at[b]],
                                  ss.at[b % 4]).wait()

        def body(c, b8, w_s4, w_g2, i_i4):
            if w_s4:
                wait_s(c - 4, (b8 - 4) % 8)
            wait_i(c, b8)
            issue_g(c, b8)
            if i_i4:
                issue_i(c + 4, (b8 + 4) % 8)
            if w_g2:
                wait_g(c - 2, (b8 - 2) % 8)
                issue_s(c - 2, (b8 - 2) % 8)

        _pipeline2(nch, body, issue_i)
        for cc in (nch - 2, nch - 1):
            wait_g(cc, cc % 8)
            issue_s(cc, cc % 8)
        for cc in range(nch - 4, nch):
            wait_s(cc, cc % 8)

        if tail:
            pltpu.sync_copy(edge_hbm.at[0, pl.ds(base + nch * CH, tail)], srct)
            pltpu.sync_copy(edge_hbm.at[1, pl.ds(base + nch * CH, tail)], dstt)
            pltpu.sync_copy(g2sh.at[srct], rowst)
            pltpu.sync_copy(rowst, accsh.at[dstt], add=True)

        plsc.subcore_barrier()
        pltpu.sync_copy(accsh.at[pl.ds(r0, zp)], stage)
        pltpu.sync_copy(stage, out_hbm.at[cid, pl.ds(r0, zp)])

    return k


def _mm_body(x_ref, w1_ref, hm_ref):
    hm_ref[...] = jnp.dot(x_ref[...], w1_ref[...],
                          preferred_element_type=jnp.float32)


def _tc3_body(aggp_ref, g2_ref, dis_ref, b2_ref, out_ref):
    s = aggp_ref[0] + aggp_ref[1] + g2_ref[...]
    out_ref[...] = jax.nn.sigmoid(dis_ref[...] * s + b2_ref[0, 0])


def kernel(x, edge_index, W1, b1, W2, b2):
    n, d = x.shape
    h = W1.shape[1]
    e = edge_index.shape[1]

    blk = 1024
    npad = ((n + 1 + blk - 1) // blk) * blk
    grid = npad // blk

    xp = jnp.pad(x, ((0, npad - n), (0, 0)))

    degp = _deg_kernel(npad, e)(edge_index)

    hm = pl.pallas_call(
        _mm_body,
        grid=(grid,),
        in_specs=[
            pl.BlockSpec((blk, d), lambda i: (i, 0)),
            pl.BlockSpec((d, h), lambda i: (0, 0)),
        ],
        out_specs=pl.BlockSpec((blk, h), lambda i: (i, 0)),
        out_shape=jax.ShapeDtypeStruct((npad, h), jnp.float32),
    )(xp, W1)

    accp, dis = _agg1_kernel(npad, e, h)(
        hm.reshape(npad * h), degp, edge_index)

    agg2p, g2 = _layer2_kernel(npad, e, h)(
        accp.reshape(NC, npad * h), dis, b1, W2.reshape(h), edge_index)

    out = pl.pallas_call(
        _tc3_body,
        grid=(grid,),
        in_specs=[
            pl.BlockSpec((NC, blk), lambda i: (0, i)),
            pl.BlockSpec((blk,), lambda i: (i,)),
            pl.BlockSpec((blk,), lambda i: (i,)),
            pl.BlockSpec((1, 1), lambda i: (0, 0)),
        ],
        out_specs=pl.BlockSpec((blk,), lambda i: (i,)),
        out_shape=jax.ShapeDtypeStruct((n,), jnp.float32),
    )(agg2p, g2, dis, b2.reshape(1, 1))

    return out.reshape(n, 1)
